# baseline (speedup 1.0000x reference)
import jax
import jax.numpy as jnp
from jax import lax
from jax.experimental import pallas as pl
from jax.experimental.pallas import tpu as pltpu
from jax.experimental.pallas import tpu_sc as plsc

N_NODES = 50000
N_EDGES = 800000
D_NODE = 64
D_HALF = 32
N_LAYERS = 3
N_GRAPHS = 128

NC = 2
NS = 16
LANES = 16

CHUNK = 32
N_CHUNKS = N_EDGES // CHUNK
ROWS_PER_TILE = 3120
ROWS_LAST = N_NODES - (NS - 1) * ROWS_PER_TILE

N_PACK = N_NODES // 4
P_ROWS_PER_TILE = 768
P_ROWS_LAST = N_PACK - (NS - 1) * P_ROWS_PER_TILE

POOL_ROWS = 784
POOL_LAST = N_PACK - (NS - 1) * POOL_ROWS
POOL_CK = 112
POOL_TAIL = POOL_LAST - 6 * POOL_CK

_BLK = 1000
_GRID_N = N_NODES // _BLK
_GRID_E = N_EDGES // _BLK


def _sigmoid16(u):
    eu = jnp.exp(-jnp.abs(u))
    s = 1.0 / (1.0 + eu)
    return jnp.where(u >= 0.0, s, eu * s)


def _softplus16(v):
    ev = jnp.exp(-jnp.abs(v))
    t = ev / (2.0 + ev)
    t2 = t * t
    p = 2.0 * t * (1.0 + t2 * (1.0 / 3.0 + t2 * (0.2 + t2 * (1.0 / 7.0))))
    return jnp.maximum(v, 0.0) + p


def _make_edge_kernel(layer):
    ea_l0 = layer * NC * N_EDGES

    def body(td_ref, ts_ref, ea_ref, dst_ref, src_ref, z_ref, agg_ref,
             agg_sh, dstb, growb, gdst, gsrc, dbuf, sbuf, eabuf, mbuf,
             sem1, sem2, sem3):
        c = lax.axis_index("c")
        s = lax.axis_index("s")
        c_n = c * N_NODES
        r0 = pl.multiple_of(s * P_ROWS_PER_TILE, 8)

        @pl.when(s < NS - 1)
        def _():
            pltpu.sync_copy(z_ref.at[pl.ds(r0, P_ROWS_PER_TILE)],
                            agg_sh.at[pl.ds(r0, P_ROWS_PER_TILE)])

        @pl.when(s == NS - 1)
        def _():
            pltpu.sync_copy(z_ref.at[pl.ds(r0, P_ROWS_LAST)],
                            agg_sh.at[pl.ds(r0, P_ROWS_LAST)])

        plsc.subcore_barrier()

        base_chunks = N_CHUNKS // NS
        extra = N_CHUNKS - NS * base_chunks
        k0 = s * base_chunks + jnp.minimum(s, extra)
        k1 = k0 + base_chunks + jnp.where(s < extra, 1, 0)
        zv = jnp.zeros((LANES,), jnp.float32)

        def chunk_body(k, carry):
            e0 = pl.multiple_of(k * CHUNK, CHUNK)
            ea_start = pl.multiple_of(ea_l0 + c * N_EDGES + e0, CHUNK)
            ea_cp = pltpu.async_copy(ea_ref.at[pl.ds(ea_start, CHUNK)],
                                     eabuf, sem3)
            pltpu.sync_copy(dst_ref.at[pl.ds(e0, CHUNK)],
                            dstb.at[pl.ds(0, CHUNK)])
            pltpu.sync_copy(src_ref.at[pl.ds(e0, CHUNK)], gsrc)
            for q in range(CHUNK // LANES):
                sl = pl.ds(q * LANES, LANES)
                dv = dstb[sl]
                gdst[sl] = dv + c_n
                growb[sl] = dv >> 2
                gsrc[sl] = gsrc[sl] + c_n
            g1 = pltpu.async_copy(td_ref.at[gdst], dbuf, sem1)
            g2 = pltpu.async_copy(ts_ref.at[gsrc], sbuf, sem2)
            g1.wait()
            g2.wait()
            ea_cp.wait()

            def edge_body(e, carry2):
                off = (dstb[pl.ds(e, LANES)][0] & 3) * D_HALF
                for q in range(8):
                    mbuf[e, pl.ds(q * LANES, LANES)] = zv
                for j in range(2):
                    slu = pl.ds(j * LANES, LANES)
                    slv = pl.ds(D_HALF + j * LANES, LANES)
                    u = dbuf[e, slu] + sbuf[e, slu] + eabuf[e, slu]
                    v = dbuf[e, slv] + sbuf[e, slv] + eabuf[e, slv]
                    m = _sigmoid16(u) * _softplus16(v)
                    mbuf[e, pl.ds(off + j * LANES, LANES)] = m
                return carry2

            lax.fori_loop(0, CHUNK, edge_body, 0)
            pltpu.sync_copy(mbuf, agg_sh.at[growb], add=True)
            return carry

        lax.fori_loop(k0, k1, chunk_body, 0)
        plsc.subcore_barrier()

        @pl.when(s < NS - 1)
        def _():
            pltpu.sync_copy(agg_sh.at[pl.ds(r0, P_ROWS_PER_TILE)],
                            agg_ref.at[c, pl.ds(r0, P_ROWS_PER_TILE)])

        @pl.when(s == NS - 1)
        def _():
            pltpu.sync_copy(agg_sh.at[pl.ds(r0, P_ROWS_LAST)],
                            agg_ref.at[c, pl.ds(r0, P_ROWS_LAST)])

    mesh = plsc.VectorSubcoreMesh(core_axis_name="c", subcore_axis_name="s",
                                  num_cores=NC, num_subcores=NS)
    return pl.kernel(
        body,
        out_type=jax.ShapeDtypeStruct((NC, N_PACK, 128), jnp.float32),
        mesh=mesh,
        compiler_params=pltpu.CompilerParams(use_tc_tiling_on_sc=False),
        scratch_types=[
            pltpu.VMEM_SHARED((N_PACK, 128), jnp.float32),
            pltpu.VMEM((CHUNK + LANES,), jnp.int32),
            pltpu.VMEM((CHUNK,), jnp.int32),
            pltpu.VMEM((CHUNK,), jnp.int32),
            pltpu.VMEM((CHUNK,), jnp.int32),
            pltpu.VMEM((CHUNK, D_NODE), jnp.float32),
            pltpu.VMEM((CHUNK, D_NODE), jnp.float32),
            pltpu.VMEM((CHUNK, D_NODE), jnp.float32),
            pltpu.VMEM((CHUNK, 128), jnp.float32),
            pltpu.SemaphoreType.DMA,
            pltpu.SemaphoreType.DMA,
            pltpu.SemaphoreType.DMA,
        ],
    )


def _pool_body(h2_ref, batch_ref, pp_ref, hbuf, bbuf, outb, sem):
    c = lax.axis_index("c")
    s = lax.axis_index("s")
    pr0 = pl.multiple_of(s * POOL_ROWS, 8)
    nb0 = pl.multiple_of(s * POOL_ROWS * 4, 8)
    neg = jnp.full((LANES,), -jnp.inf, jnp.float32)

    @pl.when(s < NS - 1)
    def _():
        pltpu.sync_copy(batch_ref.at[pl.ds(nb0, POOL_ROWS * 4)],
                        bbuf.at[pl.ds(0, POOL_ROWS * 4)])

    @pl.when(s == NS - 1)
    def _():
        pltpu.sync_copy(batch_ref.at[pl.ds(nb0, POOL_LAST * 4)],
                        bbuf.at[pl.ds(0, POOL_LAST * 4)])

    def init_body(i, carry):
        for j in range(2):
            outb[i, pl.ds(j * LANES, LANES)] = neg
        return carry

    lax.fori_loop(0, N_GRAPHS, init_body, 0)

    def ck_body(k, carry):
        is_tail = jnp.logical_and(s == NS - 1, k == 6)
        row0 = pl.multiple_of(pr0 + k * POOL_CK, 8)

        @pl.when(jnp.logical_not(is_tail))
        def _():
            pltpu.sync_copy(h2_ref.at[c, pl.ds(row0, POOL_CK)], hbuf)

        @pl.when(is_tail)
        def _():
            pltpu.sync_copy(h2_ref.at[c, pl.ds(row0, POOL_TAIL)],
                            hbuf.at[pl.ds(0, POOL_TAIL)])

        rows_k = jnp.where(is_tail, POOL_TAIL, POOL_CK)

        def row_body(p, carry2):
            nl = (k * POOL_CK + p) * 4
            for q in range(4):
                bid = bbuf[pl.ds(nl + q, LANES)][0]
                for j in range(2):
                    hv = hbuf[p, pl.ds(q * D_HALF + j * LANES, LANES)]
                    slo = pl.ds(j * LANES, LANES)
                    outb[bid, slo] = jnp.maximum(outb[bid, slo], hv)
            return carry2

        lax.fori_loop(0, rows_k, row_body, 0)
        return carry

    lax.fori_loop(0, 7, ck_body, 0)
    pltpu.sync_copy(outb, pp_ref.at[c, s])


def _make_pool_kernel():
    mesh = plsc.VectorSubcoreMesh(core_axis_name="c", subcore_axis_name="s",
                                  num_cores=NC, num_subcores=NS)
    return pl.kernel(
        _pool_body,
        out_type=jax.ShapeDtypeStruct((NC, NS, N_GRAPHS, D_HALF), jnp.float32),
        mesh=mesh,
        compiler_params=pltpu.CompilerParams(use_tc_tiling_on_sc=False),
        scratch_types=[
            pltpu.VMEM((POOL_CK, 128), jnp.float32),
            pltpu.VMEM((POOL_ROWS * 4 + LANES,), jnp.int32),
            pltpu.VMEM((N_GRAPHS, D_HALF), jnp.float32),
            pltpu.SemaphoreType.DMA,
        ],
    )


def _emb_tables_body(x_ref, wemb_ref, bemb_ref, wd_ref, ws_ref,
                     h_ref, td_ref, ts_ref):
    hb = jnp.dot(x_ref[...], wemb_ref[...],
                 preferred_element_type=jnp.float32) + bemb_ref[...]
    h_ref[...] = hb
    for cc in range(NC):
        td_ref[cc] = jnp.dot(hb, wd_ref[cc], preferred_element_type=jnp.float32)
        ts_ref[cc] = jnp.dot(hb, ws_ref[cc], preferred_element_type=jnp.float32)


def _ea_body(ea_ref, wea_ref, bea_ref, out_ref):
    eb = ea_ref[...]
    for l in range(N_LAYERS):
        for cc in range(NC):
            out_ref[l, cc] = (jnp.dot(eb, wea_ref[l, cc],
                                      preferred_element_type=jnp.float32)
                              + bea_ref[l, cc])


def _reduce_body(agg_ref, out_ref):
    i = pl.program_id(0)

    @pl.when(i == 0)
    def _():
        out_ref[...] = jnp.zeros_like(out_ref)

    a0 = agg_ref[0]
    a1 = agg_ref[1]
    s0 = jnp.sum(a0, axis=0)
    s1 = jnp.sum(a1, axis=0)
    q0 = jnp.sum(a0 * a0, axis=0)
    q1 = jnp.sum(a1 * a1, axis=0)
    row0 = jnp.concatenate([s0, s1])[None, :]
    row1 = jnp.concatenate([q0, q1])[None, :]
    pad = jnp.zeros((6, D_NODE), jnp.float32)
    out_ref[...] += jnp.concatenate([row0, row1, pad], axis=0)


def _bn_stats(sums_ref, gamma_ref, beta_ref):
    mu = sums_ref[0:1, :] * (1.0 / N_NODES)
    msq = sums_ref[1:2, :] * (1.0 / N_NODES)
    var = msq - mu * mu
    inv = gamma_ref[...] / jnp.sqrt(var + 1e-5)
    shift = beta_ref[...] - mu * inv
    return inv, shift


def _bn_tables_body(h_ref, agg_ref, sums_ref, gamma_ref, beta_ref,
                    wd_ref, ws_ref, hn_ref, td_ref, ts_ref):
    inv, shift = _bn_stats(sums_ref, gamma_ref, beta_ref)
    c0 = h_ref[:, 0:D_HALF] + agg_ref[0] * inv[:, 0:D_HALF] + shift[:, 0:D_HALF]
    c1 = h_ref[:, D_HALF:] + agg_ref[1] * inv[:, D_HALF:] + shift[:, D_HALF:]
    hn = jnp.concatenate([c0, c1], axis=1)
    hn_ref[...] = hn
    for cc in range(NC):
        td_ref[cc] = jnp.dot(hn, wd_ref[cc], preferred_element_type=jnp.float32)
        ts_ref[cc] = jnp.dot(hn, ws_ref[cc], preferred_element_type=jnp.float32)


def _bn_final_body(h_ref, agg_ref, sums_ref, gamma_ref, beta_ref, h2_ref):
    inv, shift = _bn_stats(sums_ref, gamma_ref, beta_ref)
    h2_ref[0] = (h_ref[:, 0:D_HALF] + agg_ref[0] * inv[:, 0:D_HALF]
                 + shift[:, 0:D_HALF])
    h2_ref[1] = (h_ref[:, D_HALF:] + agg_ref[1] * inv[:, D_HALF:]
                 + shift[:, D_HALF:])


def _head_body(pp_ref, wfc_ref, bfc_ref, wout_ref, bout_ref, out_ref):
    p0 = jnp.max(pp_ref[0], axis=0)
    p1 = jnp.max(pp_ref[1], axis=0)
    pooled = jnp.concatenate([p0, p1], axis=1)
    t = jnp.dot(pooled, wfc_ref[...],
                preferred_element_type=jnp.float32) + bfc_ref[...]
    sp = jnp.maximum(t, 0.0) + jnp.log(1.0 + jnp.exp(-jnp.abs(t)))
    out_ref[...] = jnp.dot(sp, wout_ref[...],
                           preferred_element_type=jnp.float32) + bout_ref[...]


def _row_spec(shape):
    nd = len(shape)
    return pl.BlockSpec(shape, lambda i: (0,) * nd)


def kernel(x, edge_index, edge_attr, batch, W_emb, b_emb, Wf, bf, Ws, bs,
           gamma, beta, W_fc, b_fc, W_out, b_out):
    f32 = jnp.float32
    src = edge_index[0]
    dst = edge_index[1]

    half = lambda w, c: w[:, c * D_HALF:(c + 1) * D_HALF]
    WD = jnp.stack([jnp.stack([
        jnp.concatenate([half(Wf[l][0:64], c), half(Ws[l][0:64], c)], axis=1)
        for c in range(NC)]) for l in range(N_LAYERS)])
    WS = jnp.stack([jnp.stack([
        jnp.concatenate([half(Wf[l][64:128], c), half(Ws[l][64:128], c)], axis=1)
        for c in range(NC)]) for l in range(N_LAYERS)])
    WEA = jnp.stack([jnp.stack([
        jnp.concatenate([half(Wf[l][128:144], c), half(Ws[l][128:144], c)],
                        axis=1)
        for c in range(NC)]) for l in range(N_LAYERS)])
    BEA = jnp.stack([jnp.stack([
        jnp.concatenate([half(bf[l][None], c)[0], half(bs[l][None], c)[0]])
        for c in range(NC)]) for l in range(N_LAYERS)])[:, :, None, :]

    zeros_n = jnp.zeros((N_PACK, 128), f32)

    ea_all = pl.pallas_call(
        _ea_body,
        grid=(_GRID_E,),
        in_specs=[
            pl.BlockSpec((_BLK, 16), lambda i: (i, 0)),
            _row_spec((N_LAYERS, NC, 16, D_NODE)),
            _row_spec((N_LAYERS, NC, 1, D_NODE)),
        ],
        out_specs=pl.BlockSpec((N_LAYERS, NC, _BLK, D_NODE),
                               lambda i: (0, 0, i, 0)),
        out_shape=jax.ShapeDtypeStruct((N_LAYERS, NC, N_EDGES, D_NODE), f32),
    )(edge_attr, WEA, BEA)
    ea_flat = ea_all.reshape(N_LAYERS * NC * N_EDGES, D_NODE)

    h, td, ts = pl.pallas_call(
        _emb_tables_body,
        grid=(_GRID_N,),
        in_specs=[
            pl.BlockSpec((_BLK, 128), lambda i: (i, 0)),
            _row_spec((128, D_NODE)),
            _row_spec((1, D_NODE)),
            _row_spec((NC, D_NODE, D_NODE)),
            _row_spec((NC, D_NODE, D_NODE)),
        ],
        out_specs=[
            pl.BlockSpec((_BLK, D_NODE), lambda i: (i, 0)),
            pl.BlockSpec((NC, _BLK, D_NODE), lambda i: (0, i, 0)),
            pl.BlockSpec((NC, _BLK, D_NODE), lambda i: (0, i, 0)),
        ],
        out_shape=[
            jax.ShapeDtypeStruct((N_NODES, D_NODE), f32),
            jax.ShapeDtypeStruct((NC, N_NODES, D_NODE), f32),
            jax.ShapeDtypeStruct((NC, N_NODES, D_NODE), f32),
        ],
    )(x, W_emb, b_emb[None, :], WD[0], WS[0])

    reduce_call = pl.pallas_call(
        _reduce_body,
        grid=(_GRID_N,),
        in_specs=[pl.BlockSpec((NC, _BLK, D_HALF), lambda i: (0, i, 0))],
        out_specs=pl.BlockSpec((8, D_NODE), lambda i: (0, 0)),
        out_shape=jax.ShapeDtypeStruct((8, D_NODE), f32),
    )

    h2 = None
    for l in range(N_LAYERS):
        edge_call = _make_edge_kernel(l)
        agg2 = edge_call(td.reshape(NC * N_NODES, D_NODE),
                         ts.reshape(NC * N_NODES, D_NODE),
                         ea_flat, dst, src, zeros_n)
        agg2 = agg2.reshape(NC, N_NODES, D_HALF)
        sums = reduce_call(agg2)
        if l < N_LAYERS - 1:
            h, td, ts = pl.pallas_call(
                _bn_tables_body,
                grid=(_GRID_N,),
                in_specs=[
                    pl.BlockSpec((_BLK, D_NODE), lambda i: (i, 0)),
                    pl.BlockSpec((NC, _BLK, D_HALF), lambda i: (0, i, 0)),
                    _row_spec((8, D_NODE)),
                    _row_spec((1, D_NODE)),
                    _row_spec((1, D_NODE)),
                    _row_spec((NC, D_NODE, D_NODE)),
                    _row_spec((NC, D_NODE, D_NODE)),
                ],
                out_specs=[
                    pl.BlockSpec((_BLK, D_NODE), lambda i: (i, 0)),
                    pl.BlockSpec((NC, _BLK, D_NODE), lambda i: (0, i, 0)),
                    pl.BlockSpec((NC, _BLK, D_NODE), lambda i: (0, i, 0)),
                ],
                out_shape=[
                    jax.ShapeDtypeStruct((N_NODES, D_NODE), f32),
                    jax.ShapeDtypeStruct((NC, N_NODES, D_NODE), f32),
                    jax.ShapeDtypeStruct((NC, N_NODES, D_NODE), f32),
                ],
            )(h, agg2, sums, gamma[l][None, :], beta[l][None, :],
              WD[l + 1], WS[l + 1])
        else:
            h2 = pl.pallas_call(
                _bn_final_body,
                grid=(_GRID_N,),
                in_specs=[
                    pl.BlockSpec((_BLK, D_NODE), lambda i: (i, 0)),
                    pl.BlockSpec((NC, _BLK, D_HALF), lambda i: (0, i, 0)),
                    _row_spec((8, D_NODE)),
                    _row_spec((1, D_NODE)),
                    _row_spec((1, D_NODE)),
                ],
                out_specs=pl.BlockSpec((NC, _BLK, D_HALF), lambda i: (0, i, 0)),
                out_shape=jax.ShapeDtypeStruct((NC, N_NODES, D_HALF), f32),
            )(h, agg2, sums, gamma[l][None, :], beta[l][None, :])

    pool_call = _make_pool_kernel()
    pp = pool_call(h2.reshape(NC, N_PACK, 128), batch)

    out = pl.pallas_call(
        _head_body,
        in_specs=[
            pl.BlockSpec((NC, NS, N_GRAPHS, D_HALF), lambda: (0, 0, 0, 0)),
            pl.BlockSpec((D_NODE, 128), lambda: (0, 0)),
            pl.BlockSpec((1, 128), lambda: (0, 0)),
            pl.BlockSpec((128, 1), lambda: (0, 0)),
            pl.BlockSpec((1, 1), lambda: (0, 0)),
        ],
        out_specs=pl.BlockSpec((N_GRAPHS, 1), lambda: (0, 0)),
        out_shape=jax.ShapeDtypeStruct((N_GRAPHS, 1), f32),
    )(pp, W_fc, b_fc[None, :], W_out, b_out[None, :])
    return out


# speedup vs baseline: 1.2327x; 1.2327x over previous
import jax
import jax.numpy as jnp
from jax import lax
from jax.experimental import pallas as pl
from jax.experimental.pallas import tpu as pltpu
from jax.experimental.pallas import tpu_sc as plsc

N_NODES = 50000
N_EDGES = 800000
D_NODE = 64
D_HALF = 32
N_LAYERS = 3
N_GRAPHS = 128

NC = 2
NS = 16
LANES = 16

CHUNK = 32
N_CHUNKS = N_EDGES // CHUNK
ROWS_PER_TILE = 3120
ROWS_LAST = N_NODES - (NS - 1) * ROWS_PER_TILE

N_PACK = N_NODES // 4
P_ROWS_PER_TILE = 768
P_ROWS_LAST = N_PACK - (NS - 1) * P_ROWS_PER_TILE

POOL_ROWS = 784
POOL_LAST = N_PACK - (NS - 1) * POOL_ROWS
POOL_CK = 112
POOL_TAIL = POOL_LAST - 6 * POOL_CK

_BLK = 1000
_GRID_N = N_NODES // _BLK
_GRID_E = N_EDGES // _BLK


def _recip(x):
    r = plsc.bitcast(jnp.asarray(0x7EF311C3, jnp.int32)
                     - plsc.bitcast(x, jnp.int32), jnp.float32)
    r = r * (2.0 - x * r)
    r = r * (2.0 - x * r)
    r = r * (2.0 - x * r)
    return r


def _sigmoid16(u):
    eu = jnp.exp(-jnp.abs(u))
    s = _recip(1.0 + eu)
    return jnp.where(u >= 0.0, s, eu * s)


def _softplus16(v):
    ev = jnp.exp(-jnp.abs(v))
    t = ev * _recip(2.0 + ev)
    t2 = t * t
    p = 2.0 * t * (1.0 + t2 * (1.0 / 3.0 + t2 * (0.2 + t2 * (1.0 / 7.0))))
    return jnp.maximum(v, 0.0) + p


EA_ROWS = CHUNK // 2
PAIRS_TOTAL = N_CHUNKS // 2


def _make_edge_kernel(layer):
    ea_l0 = (layer * NC * N_EDGES) // 2

    def body(td_ref, ts_ref, ea_ref, dst_ref, src_ref, z_ref, agg_ref,
             agg_sh, gdst, gsrc, scidx, dbuf, sbuf, eabuf, mbuf,
             sg0, sg1, si0, si1, se0, se1, sc0, sc1):
        c = lax.axis_index("c")
        s = lax.axis_index("s")
        c_n = c * N_NODES
        c_p = c * N_PACK
        r0 = pl.multiple_of(s * P_ROWS_PER_TILE, 8)
        sg = (sg0, sg1)
        si = (si0, si1)
        se = (se0, se1)
        sc = (sc0, sc1)

        @pl.when(s < NS - 1)
        def _():
            pltpu.sync_copy(z_ref.at[pl.ds(r0, P_ROWS_PER_TILE)],
                            agg_sh.at[pl.ds(r0, P_ROWS_PER_TILE)])

        @pl.when(s == NS - 1)
        def _():
            pltpu.sync_copy(z_ref.at[pl.ds(r0, P_ROWS_LAST)],
                            agg_sh.at[pl.ds(r0, P_ROWS_LAST)])

        plsc.subcore_barrier()

        base_pairs = PAIRS_TOTAL // NS
        extra = PAIRS_TOTAL - NS * base_pairs
        p0 = s * base_pairs + jnp.minimum(s, extra)
        pcnt = base_pairs + jnp.where(s < extra, 1, 0)
        k0 = p0 * 2
        zv = jnp.zeros((LANES,), jnp.float32)

        def slot(buf, b, n):
            return buf.at[pl.ds(b * n, n)]

        def issue_idx_ea(i, b):
            k = k0 + i
            e0 = pl.multiple_of(k * CHUNK, CHUNK)
            pltpu.async_copy(dst_ref.at[pl.ds(e0, CHUNK)],
                             slot(gdst, b, CHUNK), si[b])
            pltpu.async_copy(src_ref.at[pl.ds(e0, CHUNK)],
                             slot(gsrc, b, CHUNK), si[b])
            er = ea_l0 + (c * N_EDGES) // 2 + k * EA_ROWS
            pltpu.async_copy(ea_ref.at[pl.ds(er, EA_ROWS)],
                             slot(eabuf, b, EA_ROWS), se[b])

        def wait_idx(b):
            pltpu.make_async_copy(dst_ref.at[pl.ds(0, CHUNK)],
                                  slot(gdst, b, CHUNK), si[b]).wait()
            pltpu.make_async_copy(src_ref.at[pl.ds(0, CHUNK)],
                                  slot(gsrc, b, CHUNK), si[b]).wait()

        def wait_ea(b):
            pltpu.make_async_copy(ea_ref.at[pl.ds(0, EA_ROWS)],
                                  slot(eabuf, b, EA_ROWS), se[b]).wait()

        def modify_idx(b):
            for q in range(CHUNK // LANES):
                sl = pl.ds(b * CHUNK + q * LANES, LANES)
                gdst[sl] = gdst[sl] + c_n
                gsrc[sl] = gsrc[sl] + c_n

        def issue_gather(b):
            pltpu.async_copy(td_ref.at[slot(gdst, b, CHUNK)],
                             slot(dbuf, b, CHUNK), sg[b])
            pltpu.async_copy(ts_ref.at[slot(gsrc, b, CHUNK)],
                             slot(sbuf, b, CHUNK), sg[b])

        def wait_gather(b):
            pltpu.make_async_copy(td_ref.at[slot(gdst, b, CHUNK)],
                                  slot(dbuf, b, CHUNK), sg[b]).wait()
            pltpu.make_async_copy(ts_ref.at[slot(gsrc, b, CHUNK)],
                                  slot(sbuf, b, CHUNK), sg[b]).wait()

        def compute(b):
            for q in range(CHUNK // LANES):
                sl = pl.ds(b * CHUNK + q * LANES, LANES)
                scidx[b, pl.ds(q * LANES, LANES)] = (gdst[sl] >> 2) - c_p

            def edge_pair(e2, carry2):
                for t in range(2):
                    e = e2 * 2 + t
                    row = b * CHUNK + e
                    off = (gdst[pl.ds(row, LANES)][0] & 3) * D_HALF
                    earow = b * EA_ROWS + (e >> 1)
                    ecb = (e & 1) * D_NODE
                    for q in range(8):
                        mbuf[row, pl.ds(q * LANES, LANES)] = zv
                    for j in range(2):
                        slu = pl.ds(j * LANES, LANES)
                        slv = pl.ds(D_HALF + j * LANES, LANES)
                        u = (dbuf[row, slu] + sbuf[row, slu]
                             + eabuf[earow, pl.ds(ecb + j * LANES, LANES)])
                        v = (dbuf[row, slv] + sbuf[row, slv]
                             + eabuf[earow,
                                     pl.ds(ecb + D_HALF + j * LANES, LANES)])
                        m = _sigmoid16(u) * _softplus16(v)
                        mbuf[row, pl.ds(off + j * LANES, LANES)] = m
                return carry2

            lax.fori_loop(0, CHUNK // 2, edge_pair, 0)

        def issue_scatter(b):
            pltpu.async_copy(slot(mbuf, b, CHUNK), agg_sh.at[scidx.at[b]],
                             sc[b], add=True)

        def wait_scatter(b):
            pltpu.make_async_copy(slot(mbuf, b, CHUNK),
                                  agg_sh.at[scidx.at[b]], sc[b]).wait()

        issue_idx_ea(0, 0)
        issue_idx_ea(1, 1)
        wait_idx(0)
        modify_idx(0)
        issue_gather(0)

        def pair_body(kk, carry):
            for b in (0, 1):
                i = kk * 2 + b
                ob = 1 - b
                if b == 0:
                    wait_idx(ob)
                    modify_idx(ob)
                    issue_gather(ob)
                else:
                    @pl.when(kk < pcnt - 1)
                    def _():
                        wait_idx(ob)
                        modify_idx(ob)
                        issue_gather(ob)
                wait_gather(b)

                @pl.when(kk >= 1)
                def _():
                    wait_scatter(b)

                wait_ea(b)
                compute(b)
                issue_scatter(b)

                @pl.when(kk < pcnt - 1)
                def _():
                    issue_idx_ea(i + 2, b)
            return carry

        lax.fori_loop(0, pcnt, pair_body, 0)
        wait_scatter(0)
        wait_scatter(1)
        plsc.subcore_barrier()

        @pl.when(s < NS - 1)
        def _():
            pltpu.sync_copy(agg_sh.at[pl.ds(r0, P_ROWS_PER_TILE)],
                            agg_ref.at[c, pl.ds(r0, P_ROWS_PER_TILE)])

        @pl.when(s == NS - 1)
        def _():
            pltpu.sync_copy(agg_sh.at[pl.ds(r0, P_ROWS_LAST)],
                            agg_ref.at[c, pl.ds(r0, P_ROWS_LAST)])

    mesh = plsc.VectorSubcoreMesh(core_axis_name="c", subcore_axis_name="s",
                                  num_cores=NC, num_subcores=NS)
    return pl.kernel(
        body,
        out_type=jax.ShapeDtypeStruct((NC, N_PACK, 128), jnp.float32),
        mesh=mesh,
        compiler_params=pltpu.CompilerParams(use_tc_tiling_on_sc=False,
                                             needs_layout_passes=False),
        scratch_types=[
            pltpu.VMEM_SHARED((N_PACK, 128), jnp.float32),
            pltpu.VMEM((2 * CHUNK + LANES,), jnp.int32),
            pltpu.VMEM((2 * CHUNK,), jnp.int32),
            pltpu.VMEM((2, CHUNK), jnp.int32),
            pltpu.VMEM((2 * CHUNK, D_NODE), jnp.float32),
            pltpu.VMEM((2 * CHUNK, D_NODE), jnp.float32),
            pltpu.VMEM((2 * EA_ROWS, 128), jnp.float32),
            pltpu.VMEM((2 * CHUNK, 128), jnp.float32),
            pltpu.SemaphoreType.DMA,
            pltpu.SemaphoreType.DMA,
            pltpu.SemaphoreType.DMA,
            pltpu.SemaphoreType.DMA,
            pltpu.SemaphoreType.DMA,
            pltpu.SemaphoreType.DMA,
            pltpu.SemaphoreType.DMA,
            pltpu.SemaphoreType.DMA,
        ],
    )


def _pool_body(h2_ref, batch_ref, pp_ref, hbuf, bbuf, outb, sem):
    c = lax.axis_index("c")
    s = lax.axis_index("s")
    pr0 = pl.multiple_of(s * POOL_ROWS, 8)
    nb0 = pl.multiple_of(s * POOL_ROWS * 4, 8)
    neg = jnp.full((LANES,), -jnp.inf, jnp.float32)

    @pl.when(s < NS - 1)
    def _():
        pltpu.sync_copy(batch_ref.at[pl.ds(nb0, POOL_ROWS * 4)],
                        bbuf.at[pl.ds(0, POOL_ROWS * 4)])

    @pl.when(s == NS - 1)
    def _():
        pltpu.sync_copy(batch_ref.at[pl.ds(nb0, POOL_LAST * 4)],
                        bbuf.at[pl.ds(0, POOL_LAST * 4)])

    def init_body(i, carry):
        for j in range(2):
            outb[i, pl.ds(j * LANES, LANES)] = neg
        return carry

    lax.fori_loop(0, N_GRAPHS, init_body, 0)

    def ck_body(k, carry):
        is_tail = jnp.logical_and(s == NS - 1, k == 6)
        row0 = pl.multiple_of(pr0 + k * POOL_CK, 8)

        @pl.when(jnp.logical_not(is_tail))
        def _():
            pltpu.sync_copy(h2_ref.at[c, pl.ds(row0, POOL_CK)], hbuf)

        @pl.when(is_tail)
        def _():
            pltpu.sync_copy(h2_ref.at[c, pl.ds(row0, POOL_TAIL)],
                            hbuf.at[pl.ds(0, POOL_TAIL)])

        rows_k = jnp.where(is_tail, POOL_TAIL, POOL_CK)

        def row_body(p, carry2):
            nl = (k * POOL_CK + p) * 4
            for q in range(4):
                bid = bbuf[pl.ds(nl + q, LANES)][0]
                for j in range(2):
                    hv = hbuf[p, pl.ds(q * D_HALF + j * LANES, LANES)]
                    slo = pl.ds(j * LANES, LANES)
                    outb[bid, slo] = jnp.maximum(outb[bid, slo], hv)
            return carry2

        lax.fori_loop(0, rows_k, row_body, 0)
        return carry

    lax.fori_loop(0, 7, ck_body, 0)
    pltpu.sync_copy(outb, pp_ref.at[c, s])


def _make_pool_kernel():
    mesh = plsc.VectorSubcoreMesh(core_axis_name="c", subcore_axis_name="s",
                                  num_cores=NC, num_subcores=NS)
    return pl.kernel(
        _pool_body,
        out_type=jax.ShapeDtypeStruct((NC, NS, N_GRAPHS, D_HALF), jnp.float32),
        mesh=mesh,
        compiler_params=pltpu.CompilerParams(use_tc_tiling_on_sc=False),
        scratch_types=[
            pltpu.VMEM((POOL_CK, 128), jnp.float32),
            pltpu.VMEM((POOL_ROWS * 4 + LANES,), jnp.int32),
            pltpu.VMEM((N_GRAPHS, D_HALF), jnp.float32),
            pltpu.SemaphoreType.DMA,
        ],
    )


def _emb_tables_body(x_ref, wemb_ref, bemb_ref, wd_ref, ws_ref,
                     h_ref, td_ref, ts_ref):
    hb = jnp.dot(x_ref[...], wemb_ref[...],
                 preferred_element_type=jnp.float32) + bemb_ref[...]
    h_ref[...] = hb
    for cc in range(NC):
        td_ref[cc] = jnp.dot(hb, wd_ref[cc], preferred_element_type=jnp.float32)
        ts_ref[cc] = jnp.dot(hb, ws_ref[cc], preferred_element_type=jnp.float32)


def _ea_body(ea_ref, wea_ref, bea_ref, out_ref):
    eb = ea_ref[...]
    for l in range(N_LAYERS):
        for cc in range(NC):
            out_ref[l, cc] = (jnp.dot(eb, wea_ref[l, cc],
                                      preferred_element_type=jnp.float32)
                              + bea_ref[l, cc])


def _reduce_body(agg_ref, out_ref):
    i = pl.program_id(0)

    @pl.when(i == 0)
    def _():
        out_ref[...] = jnp.zeros_like(out_ref)

    a0 = agg_ref[0]
    a1 = agg_ref[1]
    s0 = jnp.sum(a0, axis=0)
    s1 = jnp.sum(a1, axis=0)
    q0 = jnp.sum(a0 * a0, axis=0)
    q1 = jnp.sum(a1 * a1, axis=0)
    row0 = jnp.concatenate([s0, s1])[None, :]
    row1 = jnp.concatenate([q0, q1])[None, :]
    pad = jnp.zeros((6, D_NODE), jnp.float32)
    out_ref[...] += jnp.concatenate([row0, row1, pad], axis=0)


def _bn_stats(sums_ref, gamma_ref, beta_ref):
    mu = sums_ref[0:1, :] * (1.0 / N_NODES)
    msq = sums_ref[1:2, :] * (1.0 / N_NODES)
    var = msq - mu * mu
    inv = gamma_ref[...] / jnp.sqrt(var + 1e-5)
    shift = beta_ref[...] - mu * inv
    return inv, shift


def _bn_tables_body(h_ref, agg_ref, sums_ref, gamma_ref, beta_ref,
                    wd_ref, ws_ref, hn_ref, td_ref, ts_ref):
    inv, shift = _bn_stats(sums_ref, gamma_ref, beta_ref)
    c0 = h_ref[:, 0:D_HALF] + agg_ref[0] * inv[:, 0:D_HALF] + shift[:, 0:D_HALF]
    c1 = h_ref[:, D_HALF:] + agg_ref[1] * inv[:, D_HALF:] + shift[:, D_HALF:]
    hn = jnp.concatenate([c0, c1], axis=1)
    hn_ref[...] = hn
    for cc in range(NC):
        td_ref[cc] = jnp.dot(hn, wd_ref[cc], preferred_element_type=jnp.float32)
        ts_ref[cc] = jnp.dot(hn, ws_ref[cc], preferred_element_type=jnp.float32)


def _bn_final_body(h_ref, agg_ref, sums_ref, gamma_ref, beta_ref, h2_ref):
    inv, shift = _bn_stats(sums_ref, gamma_ref, beta_ref)
    h2_ref[0] = (h_ref[:, 0:D_HALF] + agg_ref[0] * inv[:, 0:D_HALF]
                 + shift[:, 0:D_HALF])
    h2_ref[1] = (h_ref[:, D_HALF:] + agg_ref[1] * inv[:, D_HALF:]
                 + shift[:, D_HALF:])


def _head_body(pp_ref, wfc_ref, bfc_ref, wout_ref, bout_ref, out_ref):
    p0 = jnp.max(pp_ref[0], axis=0)
    p1 = jnp.max(pp_ref[1], axis=0)
    pooled = jnp.concatenate([p0, p1], axis=1)
    t = jnp.dot(pooled, wfc_ref[...],
                preferred_element_type=jnp.float32) + bfc_ref[...]
    sp = jnp.maximum(t, 0.0) + jnp.log(1.0 + jnp.exp(-jnp.abs(t)))
    out_ref[...] = jnp.dot(sp, wout_ref[...],
                           preferred_element_type=jnp.float32) + bout_ref[...]


def _row_spec(shape):
    nd = len(shape)
    return pl.BlockSpec(shape, lambda i: (0,) * nd)


def kernel(x, edge_index, edge_attr, batch, W_emb, b_emb, Wf, bf, Ws, bs,
           gamma, beta, W_fc, b_fc, W_out, b_out):
    f32 = jnp.float32
    src = edge_index[0]
    dst = edge_index[1]

    half = lambda w, c: w[:, c * D_HALF:(c + 1) * D_HALF]
    WD = jnp.stack([jnp.stack([
        jnp.concatenate([half(Wf[l][0:64], c), half(Ws[l][0:64], c)], axis=1)
        for c in range(NC)]) for l in range(N_LAYERS)])
    WS = jnp.stack([jnp.stack([
        jnp.concatenate([half(Wf[l][64:128], c), half(Ws[l][64:128], c)], axis=1)
        for c in range(NC)]) for l in range(N_LAYERS)])
    WEA = jnp.stack([jnp.stack([
        jnp.concatenate([half(Wf[l][128:144], c), half(Ws[l][128:144], c)],
                        axis=1)
        for c in range(NC)]) for l in range(N_LAYERS)])
    BEA = jnp.stack([jnp.stack([
        jnp.concatenate([half(bf[l][None], c)[0], half(bs[l][None], c)[0]])
        for c in range(NC)]) for l in range(N_LAYERS)])[:, :, None, :]

    zeros_n = jnp.zeros((N_PACK, 128), f32)

    ea_all = pl.pallas_call(
        _ea_body,
        grid=(_GRID_E,),
        in_specs=[
            pl.BlockSpec((_BLK, 16), lambda i: (i, 0)),
            _row_spec((N_LAYERS, NC, 16, D_NODE)),
            _row_spec((N_LAYERS, NC, 1, D_NODE)),
        ],
        out_specs=pl.BlockSpec((N_LAYERS, NC, _BLK, D_NODE),
                               lambda i: (0, 0, i, 0)),
        out_shape=jax.ShapeDtypeStruct((N_LAYERS, NC, N_EDGES, D_NODE), f32),
    )(edge_attr, WEA, BEA)
    ea_pack = ea_all.reshape(N_LAYERS * NC * N_EDGES // 2, 128)

    h, td, ts = pl.pallas_call(
        _emb_tables_body,
        grid=(_GRID_N,),
        in_specs=[
            pl.BlockSpec((_BLK, 128), lambda i: (i, 0)),
            _row_spec((128, D_NODE)),
            _row_spec((1, D_NODE)),
            _row_spec((NC, D_NODE, D_NODE)),
            _row_spec((NC, D_NODE, D_NODE)),
        ],
        out_specs=[
            pl.BlockSpec((_BLK, D_NODE), lambda i: (i, 0)),
            pl.BlockSpec((NC, _BLK, D_NODE), lambda i: (0, i, 0)),
            pl.BlockSpec((NC, _BLK, D_NODE), lambda i: (0, i, 0)),
        ],
        out_shape=[
            jax.ShapeDtypeStruct((N_NODES, D_NODE), f32),
            jax.ShapeDtypeStruct((NC, N_NODES, D_NODE), f32),
            jax.ShapeDtypeStruct((NC, N_NODES, D_NODE), f32),
        ],
    )(x, W_emb, b_emb[None, :], WD[0], WS[0])

    reduce_call = pl.pallas_call(
        _reduce_body,
        grid=(_GRID_N,),
        in_specs=[pl.BlockSpec((NC, _BLK, D_HALF), lambda i: (0, i, 0))],
        out_specs=pl.BlockSpec((8, D_NODE), lambda i: (0, 0)),
        out_shape=jax.ShapeDtypeStruct((8, D_NODE), f32),
    )

    h2 = None
    for l in range(N_LAYERS):
        edge_call = _make_edge_kernel(l)
        agg2 = edge_call(td.reshape(NC * N_NODES, D_NODE),
                         ts.reshape(NC * N_NODES, D_NODE),
                         ea_pack, dst, src, zeros_n)
        agg2 = agg2.reshape(NC, N_NODES, D_HALF)
        sums = reduce_call(agg2)
        if l < N_LAYERS - 1:
            h, td, ts = pl.pallas_call(
                _bn_tables_body,
                grid=(_GRID_N,),
                in_specs=[
                    pl.BlockSpec((_BLK, D_NODE), lambda i: (i, 0)),
                    pl.BlockSpec((NC, _BLK, D_HALF), lambda i: (0, i, 0)),
                    _row_spec((8, D_NODE)),
                    _row_spec((1, D_NODE)),
                    _row_spec((1, D_NODE)),
                    _row_spec((NC, D_NODE, D_NODE)),
                    _row_spec((NC, D_NODE, D_NODE)),
                ],
                out_specs=[
                    pl.BlockSpec((_BLK, D_NODE), lambda i: (i, 0)),
                    pl.BlockSpec((NC, _BLK, D_NODE), lambda i: (0, i, 0)),
                    pl.BlockSpec((NC, _BLK, D_NODE), lambda i: (0, i, 0)),
                ],
                out_shape=[
                    jax.ShapeDtypeStruct((N_NODES, D_NODE), f32),
                    jax.ShapeDtypeStruct((NC, N_NODES, D_NODE), f32),
                    jax.ShapeDtypeStruct((NC, N_NODES, D_NODE), f32),
                ],
            )(h, agg2, sums, gamma[l][None, :], beta[l][None, :],
              WD[l + 1], WS[l + 1])
        else:
            h2 = pl.pallas_call(
                _bn_final_body,
                grid=(_GRID_N,),
                in_specs=[
                    pl.BlockSpec((_BLK, D_NODE), lambda i: (i, 0)),
                    pl.BlockSpec((NC, _BLK, D_HALF), lambda i: (0, i, 0)),
                    _row_spec((8, D_NODE)),
                    _row_spec((1, D_NODE)),
                    _row_spec((1, D_NODE)),
                ],
                out_specs=pl.BlockSpec((NC, _BLK, D_HALF), lambda i: (0, i, 0)),
                out_shape=jax.ShapeDtypeStruct((NC, N_NODES, D_HALF), f32),
            )(h, agg2, sums, gamma[l][None, :], beta[l][None, :])

    pool_call = _make_pool_kernel()
    pp = pool_call(h2.reshape(NC, N_PACK, 128), batch)

    out = pl.pallas_call(
        _head_body,
        in_specs=[
            pl.BlockSpec((NC, NS, N_GRAPHS, D_HALF), lambda: (0, 0, 0, 0)),
            pl.BlockSpec((D_NODE, 128), lambda: (0, 0)),
            pl.BlockSpec((1, 128), lambda: (0, 0)),
            pl.BlockSpec((128, 1), lambda: (0, 0)),
            pl.BlockSpec((1, 1), lambda: (0, 0)),
        ],
        out_specs=pl.BlockSpec((N_GRAPHS, 1), lambda: (0, 0)),
        out_shape=jax.ShapeDtypeStruct((N_GRAPHS, 1), f32),
    )(pp, W_fc, b_fc[None, :], W_out, b_out[None, :])
    return out


# speedup vs baseline: 2.3916x; 1.9401x over previous
import jax
import jax.numpy as jnp
from jax import lax
from jax.experimental import pallas as pl
from jax.experimental.pallas import tpu as pltpu
from jax.experimental.pallas import tpu_sc as plsc

N_NODES = 50000
N_EDGES = 800000
D_NODE = 64
D_HALF = 32
N_LAYERS = 3
N_GRAPHS = 128

NC = 2
NS = 16
LANES = 16

CHUNK = 32
N_CHUNKS = N_EDGES // CHUNK
ROWS_PER_TILE = 3120
ROWS_LAST = N_NODES - (NS - 1) * ROWS_PER_TILE

N_PACK = N_NODES // 4
P_ROWS_PER_TILE = 768
P_ROWS_LAST = N_PACK - (NS - 1) * P_ROWS_PER_TILE

POOL_ROWS = 784
POOL_LAST = N_PACK - (NS - 1) * POOL_ROWS
POOL_CK = 112
POOL_TAIL = POOL_LAST - 6 * POOL_CK

_BLK = 1000
_GRID_N = N_NODES // _BLK
_GRID_E = N_EDGES // _BLK


def _recip(x):
    r = plsc.bitcast(jnp.asarray(0x7EF311C3, jnp.int32)
                     - plsc.bitcast(x, jnp.int32), jnp.float32)
    r = r * (2.0 - x * r)
    r = r * (2.0 - x * r)
    r = r * (2.0 - x * r)
    return r


def _sigmoid16(u):
    eu = jnp.exp(-jnp.abs(u))
    s = _recip(1.0 + eu)
    return jnp.where(u >= 0.0, s, eu * s)


def _softplus16(v):
    ev = jnp.exp(-jnp.abs(v))
    t = ev * _recip(2.0 + ev)
    t2 = t * t
    p = 2.0 * t * (1.0 + t2 * (1.0 / 3.0 + t2 * (0.2 + t2 * (1.0 / 7.0))))
    return jnp.maximum(v, 0.0) + p


EA_ROWS = CHUNK // 2
PAIRS_TOTAL = N_CHUNKS // 2


def _make_edge_kernel(layer):
    ea_l0 = (layer * NC * N_EDGES) // 2

    def body(td_ref, ts_ref, ea_ref, dst_ref, src_ref, z_ref, agg_ref,
             agg_sh, gdst, gsrc, scidx, dbuf, sbuf, eabuf, mbuf,
             sg0, sg1, si0, si1, se0, se1, sc0, sc1):
        c = lax.axis_index("c")
        s = lax.axis_index("s")
        c_n = c * N_NODES
        c_p = c * N_PACK
        r0 = pl.multiple_of(s * P_ROWS_PER_TILE, 8)
        sg = (sg0, sg1)
        si = (si0, si1)
        se = (se0, se1)
        sc = (sc0, sc1)

        @pl.when(s < NS - 1)
        def _():
            pltpu.sync_copy(z_ref.at[pl.ds(r0, P_ROWS_PER_TILE)],
                            agg_sh.at[pl.ds(r0, P_ROWS_PER_TILE)])

        @pl.when(s == NS - 1)
        def _():
            pltpu.sync_copy(z_ref.at[pl.ds(r0, P_ROWS_LAST)],
                            agg_sh.at[pl.ds(r0, P_ROWS_LAST)])

        plsc.subcore_barrier()

        base_pairs = PAIRS_TOTAL // NS
        extra = PAIRS_TOTAL - NS * base_pairs
        p0 = s * base_pairs + jnp.minimum(s, extra)
        pcnt = base_pairs + jnp.where(s < extra, 1, 0)
        k0 = p0 * 2
        zv = jnp.zeros((LANES,), jnp.float32)

        def slot(buf, b, n):
            return buf.at[pl.ds(b * n, n)]

        def issue_idx_ea(i, b):
            k = k0 + i
            e0 = pl.multiple_of(k * CHUNK, CHUNK)
            pltpu.async_copy(dst_ref.at[pl.ds(e0, CHUNK)],
                             slot(gdst, b, CHUNK), si[b])
            pltpu.async_copy(src_ref.at[pl.ds(e0, CHUNK)],
                             slot(gsrc, b, CHUNK), si[b])
            er = ea_l0 + (c * N_EDGES) // 2 + k * EA_ROWS
            pltpu.async_copy(ea_ref.at[pl.ds(er, EA_ROWS)],
                             slot(eabuf, b, EA_ROWS), se[b])

        def wait_idx(b):
            pltpu.make_async_copy(dst_ref.at[pl.ds(0, CHUNK)],
                                  slot(gdst, b, CHUNK), si[b]).wait()
            pltpu.make_async_copy(src_ref.at[pl.ds(0, CHUNK)],
                                  slot(gsrc, b, CHUNK), si[b]).wait()

        def wait_ea(b):
            pltpu.make_async_copy(ea_ref.at[pl.ds(0, EA_ROWS)],
                                  slot(eabuf, b, EA_ROWS), se[b]).wait()

        def modify_idx(b):
            for q in range(CHUNK // LANES):
                sl = pl.ds(b * CHUNK + q * LANES, LANES)
                gdst[sl] = gdst[sl] + c_n
                gsrc[sl] = gsrc[sl] + c_n

        def issue_gather(b):
            pltpu.async_copy(td_ref.at[slot(gdst, b, CHUNK)],
                             slot(dbuf, b, CHUNK), sg[b])
            pltpu.async_copy(ts_ref.at[slot(gsrc, b, CHUNK)],
                             slot(sbuf, b, CHUNK), sg[b])

        def wait_gather(b):
            pltpu.make_async_copy(td_ref.at[slot(gdst, b, CHUNK)],
                                  slot(dbuf, b, CHUNK), sg[b]).wait()
            pltpu.make_async_copy(ts_ref.at[slot(gsrc, b, CHUNK)],
                                  slot(sbuf, b, CHUNK), sg[b]).wait()

        def compute(b):
            for q in range(CHUNK // LANES):
                sl = pl.ds(b * CHUNK + q * LANES, LANES)
                scidx[b, pl.ds(q * LANES, LANES)] = (gdst[sl] >> 2) - c_p

            lanes = lax.broadcasted_iota(jnp.int32, (LANES,), 0)

            @plsc.parallel_loop(0, CHUNK, unroll=2)
            def _(e):
                row = b * CHUNK + e
                rowv = jnp.zeros((LANES,), jnp.int32) + row
                dv = plsc.load_gather(gdst, [rowv])
                offv = (dv & 3) * D_HALF + lanes
                earow = b * EA_ROWS + (e >> 1)
                ecb = (e & 1) * D_NODE
                for q in range(8):
                    mbuf[row, pl.ds(q * LANES, LANES)] = zv
                for j in range(2):
                    slu = pl.ds(j * LANES, LANES)
                    slv = pl.ds(D_HALF + j * LANES, LANES)
                    u = (dbuf[row, slu] + sbuf[row, slu]
                         + eabuf[earow, pl.ds(ecb + j * LANES, LANES)])
                    v = (dbuf[row, slv] + sbuf[row, slv]
                         + eabuf[earow,
                                 pl.ds(ecb + D_HALF + j * LANES, LANES)])
                    m = _sigmoid16(u) * _softplus16(v)
                    plsc.store_scatter(mbuf, [rowv, offv + (j * LANES)], m)

        def issue_scatter(b):
            pltpu.async_copy(slot(mbuf, b, CHUNK), agg_sh.at[scidx.at[b]],
                             sc[b], add=True)

        def wait_scatter(b):
            pltpu.make_async_copy(slot(mbuf, b, CHUNK),
                                  agg_sh.at[scidx.at[b]], sc[b]).wait()

        issue_idx_ea(0, 0)
        issue_idx_ea(1, 1)
        wait_idx(0)
        modify_idx(0)
        issue_gather(0)

        def pair_body(kk, carry):
            for b in (0, 1):
                i = kk * 2 + b
                ob = 1 - b
                if b == 0:
                    wait_idx(ob)
                    modify_idx(ob)
                    issue_gather(ob)
                else:
                    @pl.when(kk < pcnt - 1)
                    def _():
                        wait_idx(ob)
                        modify_idx(ob)
                        issue_gather(ob)
                wait_gather(b)

                @pl.when(kk >= 1)
                def _():
                    wait_scatter(b)

                wait_ea(b)
                compute(b)
                issue_scatter(b)

                @pl.when(kk < pcnt - 1)
                def _():
                    issue_idx_ea(i + 2, b)
            return carry

        lax.fori_loop(0, pcnt, pair_body, 0)
        wait_scatter(0)
        wait_scatter(1)
        plsc.subcore_barrier()

        @pl.when(s < NS - 1)
        def _():
            pltpu.sync_copy(agg_sh.at[pl.ds(r0, P_ROWS_PER_TILE)],
                            agg_ref.at[c, pl.ds(r0, P_ROWS_PER_TILE)])

        @pl.when(s == NS - 1)
        def _():
            pltpu.sync_copy(agg_sh.at[pl.ds(r0, P_ROWS_LAST)],
                            agg_ref.at[c, pl.ds(r0, P_ROWS_LAST)])

    mesh = plsc.VectorSubcoreMesh(core_axis_name="c", subcore_axis_name="s",
                                  num_cores=NC, num_subcores=NS)
    return pl.kernel(
        body,
        out_type=jax.ShapeDtypeStruct((NC, N_PACK, 128), jnp.float32),
        mesh=mesh,
        compiler_params=pltpu.CompilerParams(use_tc_tiling_on_sc=False,
                                             needs_layout_passes=False),
        scratch_types=[
            pltpu.VMEM_SHARED((N_PACK, 128), jnp.float32),
            pltpu.VMEM((2 * CHUNK + LANES,), jnp.int32),
            pltpu.VMEM((2 * CHUNK,), jnp.int32),
            pltpu.VMEM((2, CHUNK), jnp.int32),
            pltpu.VMEM((2 * CHUNK, D_NODE), jnp.float32),
            pltpu.VMEM((2 * CHUNK, D_NODE), jnp.float32),
            pltpu.VMEM((2 * EA_ROWS, 128), jnp.float32),
            pltpu.VMEM((2 * CHUNK, 128), jnp.float32),
            pltpu.SemaphoreType.DMA,
            pltpu.SemaphoreType.DMA,
            pltpu.SemaphoreType.DMA,
            pltpu.SemaphoreType.DMA,
            pltpu.SemaphoreType.DMA,
            pltpu.SemaphoreType.DMA,
            pltpu.SemaphoreType.DMA,
            pltpu.SemaphoreType.DMA,
        ],
    )


def _pool_body(h2_ref, batch_ref, pp_ref, hbuf, bbuf, outb, sem):
    c = lax.axis_index("c")
    s = lax.axis_index("s")
    pr0 = pl.multiple_of(s * POOL_ROWS, 8)
    nb0 = pl.multiple_of(s * POOL_ROWS * 4, 8)
    neg = jnp.full((LANES,), -jnp.inf, jnp.float32)

    @pl.when(s < NS - 1)
    def _():
        pltpu.sync_copy(batch_ref.at[pl.ds(nb0, POOL_ROWS * 4)],
                        bbuf.at[pl.ds(0, POOL_ROWS * 4)])

    @pl.when(s == NS - 1)
    def _():
        pltpu.sync_copy(batch_ref.at[pl.ds(nb0, POOL_LAST * 4)],
                        bbuf.at[pl.ds(0, POOL_LAST * 4)])

    def init_body(i, carry):
        for j in range(2):
            outb[i, pl.ds(j * LANES, LANES)] = neg
        return carry

    lax.fori_loop(0, N_GRAPHS, init_body, 0)

    def ck_body(k, carry):
        is_tail = jnp.logical_and(s == NS - 1, k == 6)
        row0 = pl.multiple_of(pr0 + k * POOL_CK, 8)

        @pl.when(jnp.logical_not(is_tail))
        def _():
            pltpu.sync_copy(h2_ref.at[c, pl.ds(row0, POOL_CK)], hbuf)

        @pl.when(is_tail)
        def _():
            pltpu.sync_copy(h2_ref.at[c, pl.ds(row0, POOL_TAIL)],
                            hbuf.at[pl.ds(0, POOL_TAIL)])

        rows_k = jnp.where(is_tail, POOL_TAIL, POOL_CK)

        def row_body(p, carry2):
            nl = (k * POOL_CK + p) * 4
            for q in range(4):
                bid = bbuf[pl.ds(nl + q, LANES)][0]
                for j in range(2):
                    hv = hbuf[p, pl.ds(q * D_HALF + j * LANES, LANES)]
                    slo = pl.ds(j * LANES, LANES)
                    outb[bid, slo] = jnp.maximum(outb[bid, slo], hv)
            return carry2

        lax.fori_loop(0, rows_k, row_body, 0)
        return carry

    lax.fori_loop(0, 7, ck_body, 0)
    pltpu.sync_copy(outb, pp_ref.at[c, s])


def _make_pool_kernel():
    mesh = plsc.VectorSubcoreMesh(core_axis_name="c", subcore_axis_name="s",
                                  num_cores=NC, num_subcores=NS)
    return pl.kernel(
        _pool_body,
        out_type=jax.ShapeDtypeStruct((NC, NS, N_GRAPHS, D_HALF), jnp.float32),
        mesh=mesh,
        compiler_params=pltpu.CompilerParams(use_tc_tiling_on_sc=False),
        scratch_types=[
            pltpu.VMEM((POOL_CK, 128), jnp.float32),
            pltpu.VMEM((POOL_ROWS * 4 + LANES,), jnp.int32),
            pltpu.VMEM((N_GRAPHS, D_HALF), jnp.float32),
            pltpu.SemaphoreType.DMA,
        ],
    )


def _emb_tables_body(x_ref, wemb_ref, bemb_ref, wd_ref, ws_ref,
                     h_ref, td_ref, ts_ref):
    hb = jnp.dot(x_ref[...], wemb_ref[...],
                 preferred_element_type=jnp.float32) + bemb_ref[...]
    h_ref[...] = hb
    for cc in range(NC):
        td_ref[cc] = jnp.dot(hb, wd_ref[cc], preferred_element_type=jnp.float32)
        ts_ref[cc] = jnp.dot(hb, ws_ref[cc], preferred_element_type=jnp.float32)


def _ea_body(ea_ref, wea_ref, bea_ref, out_ref):
    eb = ea_ref[...]
    for l in range(N_LAYERS):
        for cc in range(NC):
            out_ref[l, cc] = (jnp.dot(eb, wea_ref[l, cc],
                                      preferred_element_type=jnp.float32)
                              + bea_ref[l, cc])


def _reduce_body(agg_ref, out_ref):
    i = pl.program_id(0)

    @pl.when(i == 0)
    def _():
        out_ref[...] = jnp.zeros_like(out_ref)

    a0 = agg_ref[0]
    a1 = agg_ref[1]
    s0 = jnp.sum(a0, axis=0)
    s1 = jnp.sum(a1, axis=0)
    q0 = jnp.sum(a0 * a0, axis=0)
    q1 = jnp.sum(a1 * a1, axis=0)
    row0 = jnp.concatenate([s0, s1])[None, :]
    row1 = jnp.concatenate([q0, q1])[None, :]
    pad = jnp.zeros((6, D_NODE), jnp.float32)
    out_ref[...] += jnp.concatenate([row0, row1, pad], axis=0)


def _bn_stats(sums_ref, gamma_ref, beta_ref):
    mu = sums_ref[0:1, :] * (1.0 / N_NODES)
    msq = sums_ref[1:2, :] * (1.0 / N_NODES)
    var = msq - mu * mu
    inv = gamma_ref[...] / jnp.sqrt(var + 1e-5)
    shift = beta_ref[...] - mu * inv
    return inv, shift


def _bn_tables_body(h_ref, agg_ref, sums_ref, gamma_ref, beta_ref,
                    wd_ref, ws_ref, hn_ref, td_ref, ts_ref):
    inv, shift = _bn_stats(sums_ref, gamma_ref, beta_ref)
    c0 = h_ref[:, 0:D_HALF] + agg_ref[0] * inv[:, 0:D_HALF] + shift[:, 0:D_HALF]
    c1 = h_ref[:, D_HALF:] + agg_ref[1] * inv[:, D_HALF:] + shift[:, D_HALF:]
    hn = jnp.concatenate([c0, c1], axis=1)
    hn_ref[...] = hn
    for cc in range(NC):
        td_ref[cc] = jnp.dot(hn, wd_ref[cc], preferred_element_type=jnp.float32)
        ts_ref[cc] = jnp.dot(hn, ws_ref[cc], preferred_element_type=jnp.float32)


def _bn_final_body(h_ref, agg_ref, sums_ref, gamma_ref, beta_ref, h2_ref):
    inv, shift = _bn_stats(sums_ref, gamma_ref, beta_ref)
    h2_ref[0] = (h_ref[:, 0:D_HALF] + agg_ref[0] * inv[:, 0:D_HALF]
                 + shift[:, 0:D_HALF])
    h2_ref[1] = (h_ref[:, D_HALF:] + agg_ref[1] * inv[:, D_HALF:]
                 + shift[:, D_HALF:])


def _head_body(pp_ref, wfc_ref, bfc_ref, wout_ref, bout_ref, out_ref):
    p0 = jnp.max(pp_ref[0], axis=0)
    p1 = jnp.max(pp_ref[1], axis=0)
    pooled = jnp.concatenate([p0, p1], axis=1)
    t = jnp.dot(pooled, wfc_ref[...],
                preferred_element_type=jnp.float32) + bfc_ref[...]
    sp = jnp.maximum(t, 0.0) + jnp.log(1.0 + jnp.exp(-jnp.abs(t)))
    out_ref[...] = jnp.dot(sp, wout_ref[...],
                           preferred_element_type=jnp.float32) + bout_ref[...]


def _row_spec(shape):
    nd = len(shape)
    return pl.BlockSpec(shape, lambda i: (0,) * nd)


def kernel(x, edge_index, edge_attr, batch, W_emb, b_emb, Wf, bf, Ws, bs,
           gamma, beta, W_fc, b_fc, W_out, b_out):
    f32 = jnp.float32
    src = edge_index[0]
    dst = edge_index[1]

    half = lambda w, c: w[:, c * D_HALF:(c + 1) * D_HALF]
    WD = jnp.stack([jnp.stack([
        jnp.concatenate([half(Wf[l][0:64], c), half(Ws[l][0:64], c)], axis=1)
        for c in range(NC)]) for l in range(N_LAYERS)])
    WS = jnp.stack([jnp.stack([
        jnp.concatenate([half(Wf[l][64:128], c), half(Ws[l][64:128], c)], axis=1)
        for c in range(NC)]) for l in range(N_LAYERS)])
    WEA = jnp.stack([jnp.stack([
        jnp.concatenate([half(Wf[l][128:144], c), half(Ws[l][128:144], c)],
                        axis=1)
        for c in range(NC)]) for l in range(N_LAYERS)])
    BEA = jnp.stack([jnp.stack([
        jnp.concatenate([half(bf[l][None], c)[0], half(bs[l][None], c)[0]])
        for c in range(NC)]) for l in range(N_LAYERS)])[:, :, None, :]

    zeros_n = jnp.zeros((N_PACK, 128), f32)

    ea_all = pl.pallas_call(
        _ea_body,
        grid=(_GRID_E,),
        in_specs=[
            pl.BlockSpec((_BLK, 16), lambda i: (i, 0)),
            _row_spec((N_LAYERS, NC, 16, D_NODE)),
            _row_spec((N_LAYERS, NC, 1, D_NODE)),
        ],
        out_specs=pl.BlockSpec((N_LAYERS, NC, _BLK, D_NODE),
                               lambda i: (0, 0, i, 0)),
        out_shape=jax.ShapeDtypeStruct((N_LAYERS, NC, N_EDGES, D_NODE), f32),
    )(edge_attr, WEA, BEA)
    ea_pack = ea_all.reshape(N_LAYERS * NC * N_EDGES // 2, 128)

    h, td, ts = pl.pallas_call(
        _emb_tables_body,
        grid=(_GRID_N,),
        in_specs=[
            pl.BlockSpec((_BLK, 128), lambda i: (i, 0)),
            _row_spec((128, D_NODE)),
            _row_spec((1, D_NODE)),
            _row_spec((NC, D_NODE, D_NODE)),
            _row_spec((NC, D_NODE, D_NODE)),
        ],
        out_specs=[
            pl.BlockSpec((_BLK, D_NODE), lambda i: (i, 0)),
            pl.BlockSpec((NC, _BLK, D_NODE), lambda i: (0, i, 0)),
            pl.BlockSpec((NC, _BLK, D_NODE), lambda i: (0, i, 0)),
        ],
        out_shape=[
            jax.ShapeDtypeStruct((N_NODES, D_NODE), f32),
            jax.ShapeDtypeStruct((NC, N_NODES, D_NODE), f32),
            jax.ShapeDtypeStruct((NC, N_NODES, D_NODE), f32),
        ],
    )(x, W_emb, b_emb[None, :], WD[0], WS[0])

    reduce_call = pl.pallas_call(
        _reduce_body,
        grid=(_GRID_N,),
        in_specs=[pl.BlockSpec((NC, _BLK, D_HALF), lambda i: (0, i, 0))],
        out_specs=pl.BlockSpec((8, D_NODE), lambda i: (0, 0)),
        out_shape=jax.ShapeDtypeStruct((8, D_NODE), f32),
    )

    h2 = None
    for l in range(N_LAYERS):
        edge_call = _make_edge_kernel(l)
        agg2 = edge_call(td.reshape(NC * N_NODES, D_NODE),
                         ts.reshape(NC * N_NODES, D_NODE),
                         ea_pack, dst, src, zeros_n)
        agg2 = agg2.reshape(NC, N_NODES, D_HALF)
        sums = reduce_call(agg2)
        if l < N_LAYERS - 1:
            h, td, ts = pl.pallas_call(
                _bn_tables_body,
                grid=(_GRID_N,),
                in_specs=[
                    pl.BlockSpec((_BLK, D_NODE), lambda i: (i, 0)),
                    pl.BlockSpec((NC, _BLK, D_HALF), lambda i: (0, i, 0)),
                    _row_spec((8, D_NODE)),
                    _row_spec((1, D_NODE)),
                    _row_spec((1, D_NODE)),
                    _row_spec((NC, D_NODE, D_NODE)),
                    _row_spec((NC, D_NODE, D_NODE)),
                ],
                out_specs=[
                    pl.BlockSpec((_BLK, D_NODE), lambda i: (i, 0)),
                    pl.BlockSpec((NC, _BLK, D_NODE), lambda i: (0, i, 0)),
                    pl.BlockSpec((NC, _BLK, D_NODE), lambda i: (0, i, 0)),
                ],
                out_shape=[
                    jax.ShapeDtypeStruct((N_NODES, D_NODE), f32),
                    jax.ShapeDtypeStruct((NC, N_NODES, D_NODE), f32),
                    jax.ShapeDtypeStruct((NC, N_NODES, D_NODE), f32),
                ],
            )(h, agg2, sums, gamma[l][None, :], beta[l][None, :],
              WD[l + 1], WS[l + 1])
        else:
            h2 = pl.pallas_call(
                _bn_final_body,
                grid=(_GRID_N,),
                in_specs=[
                    pl.BlockSpec((_BLK, D_NODE), lambda i: (i, 0)),
                    pl.BlockSpec((NC, _BLK, D_HALF), lambda i: (0, i, 0)),
                    _row_spec((8, D_NODE)),
                    _row_spec((1, D_NODE)),
                    _row_spec((1, D_NODE)),
                ],
                out_specs=pl.BlockSpec((NC, _BLK, D_HALF), lambda i: (0, i, 0)),
                out_shape=jax.ShapeDtypeStruct((NC, N_NODES, D_HALF), f32),
            )(h, agg2, sums, gamma[l][None, :], beta[l][None, :])

    pool_call = _make_pool_kernel()
    pp = pool_call(h2.reshape(NC, N_PACK, 128), batch)

    out = pl.pallas_call(
        _head_body,
        in_specs=[
            pl.BlockSpec((NC, NS, N_GRAPHS, D_HALF), lambda: (0, 0, 0, 0)),
            pl.BlockSpec((D_NODE, 128), lambda: (0, 0)),
            pl.BlockSpec((1, 128), lambda: (0, 0)),
            pl.BlockSpec((128, 1), lambda: (0, 0)),
            pl.BlockSpec((1, 1), lambda: (0, 0)),
        ],
        out_specs=pl.BlockSpec((N_GRAPHS, 1), lambda: (0, 0)),
        out_shape=jax.ShapeDtypeStruct((N_GRAPHS, 1), f32),
    )(pp, W_fc, b_fc[None, :], W_out, b_out[None, :])
    return out


# speedup vs baseline: 2.5368x; 1.0607x over previous
import jax
import jax.numpy as jnp
from jax import lax
from jax.experimental import pallas as pl
from jax.experimental.pallas import tpu as pltpu
from jax.experimental.pallas import tpu_sc as plsc

N_NODES = 50000
N_EDGES = 800000
D_NODE = 64
D_HALF = 32
N_LAYERS = 3
N_GRAPHS = 128

NC = 2
NS = 16
LANES = 16

CHUNK = 32
N_CHUNKS = N_EDGES // CHUNK
ROWS_PER_TILE = 3120
ROWS_LAST = N_NODES - (NS - 1) * ROWS_PER_TILE

N_PACK = N_NODES // 4
P_ROWS_PER_TILE = 768
P_ROWS_LAST = N_PACK - (NS - 1) * P_ROWS_PER_TILE

POOL_ROWS = 784
POOL_LAST = N_PACK - (NS - 1) * POOL_ROWS
POOL_CK = 112
POOL_TAIL = POOL_LAST - 6 * POOL_CK

_BLK = 1000
_GRID_N = N_NODES // _BLK
_GRID_E = N_EDGES // _BLK


def _recip(x):
    r = plsc.bitcast(jnp.asarray(0x7EF311C3, jnp.int32)
                     - plsc.bitcast(x, jnp.int32), jnp.float32)
    r = r * (2.0 - x * r)
    r = r * (2.0 - x * r)
    r = r * (2.0 - x * r)
    return r


def _sigmoid16(u):
    eu = jnp.exp(-jnp.abs(u))
    s = _recip(1.0 + eu)
    return jnp.where(u >= 0.0, s, eu * s)


def _softplus16(v):
    ev = jnp.exp(-jnp.abs(v))
    t = ev * _recip(2.0 + ev)
    t2 = t * t
    p = 2.0 * t * (1.0 + t2 * (1.0 / 3.0 + t2 * (0.2 + t2 * (1.0 / 7.0))))
    return jnp.maximum(v, 0.0) + p


EA_ROWS = CHUNK // 2
PAIRS_TOTAL = N_CHUNKS // 2


def _make_edge_kernel(layer):
    ea_plane0 = layer * NC

    def body(td_ref, ts_ref, ea_ref, dst_ref, src_ref, z_ref, agg_ref,
             agg_sh, gdst, gsrc, scidx, dbuf, sbuf, eabuf, mbuf,
             sg0, sg1, si0, si1, se0, se1, sc0, sc1):
        c = lax.axis_index("c")
        s = lax.axis_index("s")
        c_n = c * N_NODES
        c_p = c * N_PACK
        r0 = pl.multiple_of(s * P_ROWS_PER_TILE, 8)
        sg = (sg0, sg1)
        si = (si0, si1)
        se = (se0, se1)
        sc = (sc0, sc1)

        @pl.when(s < NS - 1)
        def _():
            pltpu.sync_copy(z_ref.at[pl.ds(r0, P_ROWS_PER_TILE)],
                            agg_sh.at[pl.ds(r0, P_ROWS_PER_TILE)])

        @pl.when(s == NS - 1)
        def _():
            pltpu.sync_copy(z_ref.at[pl.ds(r0, P_ROWS_LAST)],
                            agg_sh.at[pl.ds(r0, P_ROWS_LAST)])

        plsc.subcore_barrier()

        base_pairs = PAIRS_TOTAL // NS
        extra = PAIRS_TOTAL - NS * base_pairs
        p0 = s * base_pairs + jnp.minimum(s, extra)
        pcnt = base_pairs + jnp.where(s < extra, 1, 0)
        k0 = p0 * 2
        zv = jnp.zeros((LANES,), jnp.float32)

        def slot(buf, b, n):
            return buf.at[pl.ds(b * n, n)]

        def issue_idx_ea(i, b):
            k = k0 + i
            e0 = pl.multiple_of(k * CHUNK, CHUNK)
            pltpu.async_copy(dst_ref.at[pl.ds(e0, CHUNK)],
                             slot(gdst, b, CHUNK), si[b])
            pltpu.async_copy(src_ref.at[pl.ds(e0, CHUNK)],
                             slot(gsrc, b, CHUNK), si[b])
            pltpu.async_copy(
                ea_ref.at[ea_plane0 + c, pl.ds(k * EA_ROWS, EA_ROWS)],
                slot(eabuf, b, EA_ROWS), se[b])

        def wait_idx(b):
            pltpu.make_async_copy(dst_ref.at[pl.ds(0, CHUNK)],
                                  slot(gdst, b, CHUNK), si[b]).wait()
            pltpu.make_async_copy(src_ref.at[pl.ds(0, CHUNK)],
                                  slot(gsrc, b, CHUNK), si[b]).wait()

        def wait_ea(b):
            pltpu.make_async_copy(ea_ref.at[0, pl.ds(0, EA_ROWS)],
                                  slot(eabuf, b, EA_ROWS), se[b]).wait()

        def modify_idx(b):
            for q in range(CHUNK // LANES):
                sl = pl.ds(b * CHUNK + q * LANES, LANES)
                gdst[sl] = gdst[sl] + c_n
                gsrc[sl] = gsrc[sl] + c_n

        def issue_gather(b):
            pltpu.async_copy(td_ref.at[slot(gdst, b, CHUNK)],
                             slot(dbuf, b, CHUNK), sg[b])
            pltpu.async_copy(ts_ref.at[slot(gsrc, b, CHUNK)],
                             slot(sbuf, b, CHUNK), sg[b])

        def wait_gather(b):
            pltpu.make_async_copy(td_ref.at[slot(gdst, b, CHUNK)],
                                  slot(dbuf, b, CHUNK), sg[b]).wait()
            pltpu.make_async_copy(ts_ref.at[slot(gsrc, b, CHUNK)],
                                  slot(sbuf, b, CHUNK), sg[b]).wait()

        def compute(b):
            for q in range(CHUNK // LANES):
                sl = pl.ds(b * CHUNK + q * LANES, LANES)
                scidx[b, pl.ds(q * LANES, LANES)] = (gdst[sl] >> 2) - c_p

            lanes = lax.broadcasted_iota(jnp.int32, (LANES,), 0)

            @plsc.parallel_loop(0, CHUNK, unroll=4)
            def _(e):
                row = b * CHUNK + e
                rowv = jnp.zeros((LANES,), jnp.int32) + row
                dv = plsc.load_gather(gdst, [rowv])
                offv = (dv & 3) * D_HALF + lanes
                earow = b * EA_ROWS + (e >> 1)
                ecb = (e & 1) * D_NODE
                for q in range(8):
                    mbuf[row, pl.ds(q * LANES, LANES)] = zv
                for j in range(2):
                    slu = pl.ds(j * LANES, LANES)
                    slv = pl.ds(D_HALF + j * LANES, LANES)
                    u = (dbuf[row, slu] + sbuf[row, slu]
                         + eabuf[earow, pl.ds(ecb + j * LANES, LANES)])
                    v = (dbuf[row, slv] + sbuf[row, slv]
                         + eabuf[earow,
                                 pl.ds(ecb + D_HALF + j * LANES, LANES)])
                    m = _sigmoid16(u) * _softplus16(v)
                    plsc.store_scatter(mbuf, [rowv, offv + (j * LANES)], m)

        def issue_scatter(b):
            pltpu.async_copy(slot(mbuf, b, CHUNK), agg_sh.at[scidx.at[b]],
                             sc[b], add=True)

        def wait_scatter(b):
            pltpu.make_async_copy(slot(mbuf, b, CHUNK),
                                  agg_sh.at[scidx.at[b]], sc[b]).wait()

        issue_idx_ea(0, 0)
        issue_idx_ea(1, 1)
        wait_idx(0)
        modify_idx(0)
        issue_gather(0)

        def pair_body(kk, carry):
            for b in (0, 1):
                i = kk * 2 + b
                ob = 1 - b
                if b == 0:
                    wait_idx(ob)
                    modify_idx(ob)
                    issue_gather(ob)
                else:
                    @pl.when(kk < pcnt - 1)
                    def _():
                        wait_idx(ob)
                        modify_idx(ob)
                        issue_gather(ob)
                wait_gather(b)

                @pl.when(kk >= 1)
                def _():
                    wait_scatter(b)

                wait_ea(b)
                compute(b)
                issue_scatter(b)

                @pl.when(kk < pcnt - 1)
                def _():
                    issue_idx_ea(i + 2, b)
            return carry

        lax.fori_loop(0, pcnt, pair_body, 0)
        wait_scatter(0)
        wait_scatter(1)
        plsc.subcore_barrier()

        @pl.when(s < NS - 1)
        def _():
            pltpu.sync_copy(agg_sh.at[pl.ds(r0, P_ROWS_PER_TILE)],
                            agg_ref.at[c, pl.ds(r0, P_ROWS_PER_TILE)])

        @pl.when(s == NS - 1)
        def _():
            pltpu.sync_copy(agg_sh.at[pl.ds(r0, P_ROWS_LAST)],
                            agg_ref.at[c, pl.ds(r0, P_ROWS_LAST)])

    mesh = plsc.VectorSubcoreMesh(core_axis_name="c", subcore_axis_name="s",
                                  num_cores=NC, num_subcores=NS)
    return pl.kernel(
        body,
        out_type=jax.ShapeDtypeStruct((NC, N_PACK, 128), jnp.float32),
        mesh=mesh,
        compiler_params=pltpu.CompilerParams(use_tc_tiling_on_sc=False,
                                             needs_layout_passes=False),
        scratch_types=[
            pltpu.VMEM_SHARED((N_PACK, 128), jnp.float32),
            pltpu.VMEM((2 * CHUNK + LANES,), jnp.int32),
            pltpu.VMEM((2 * CHUNK,), jnp.int32),
            pltpu.VMEM((2, CHUNK), jnp.int32),
            pltpu.VMEM((2 * CHUNK, D_NODE), jnp.float32),
            pltpu.VMEM((2 * CHUNK, D_NODE), jnp.float32),
            pltpu.VMEM((2 * EA_ROWS, 128), jnp.float32),
            pltpu.VMEM((2 * CHUNK, 128), jnp.float32),
            pltpu.SemaphoreType.DMA,
            pltpu.SemaphoreType.DMA,
            pltpu.SemaphoreType.DMA,
            pltpu.SemaphoreType.DMA,
            pltpu.SemaphoreType.DMA,
            pltpu.SemaphoreType.DMA,
            pltpu.SemaphoreType.DMA,
            pltpu.SemaphoreType.DMA,
        ],
    )


def _pool_body(h2_ref, batch_ref, pp_ref, hbuf, bbuf, outb, sem):
    c = lax.axis_index("c")
    s = lax.axis_index("s")
    pr0 = pl.multiple_of(s * POOL_ROWS, 8)
    nb0 = pl.multiple_of(s * POOL_ROWS * 4, 8)
    neg = jnp.full((LANES,), -jnp.inf, jnp.float32)

    @pl.when(s < NS - 1)
    def _():
        pltpu.sync_copy(batch_ref.at[pl.ds(nb0, POOL_ROWS * 4)],
                        bbuf.at[pl.ds(0, POOL_ROWS * 4)])

    @pl.when(s == NS - 1)
    def _():
        pltpu.sync_copy(batch_ref.at[pl.ds(nb0, POOL_LAST * 4)],
                        bbuf.at[pl.ds(0, POOL_LAST * 4)])

    def init_body(i, carry):
        for j in range(2):
            outb[i, pl.ds(j * LANES, LANES)] = neg
        return carry

    lax.fori_loop(0, N_GRAPHS, init_body, 0)

    def ck_body(k, carry):
        is_tail = jnp.logical_and(s == NS - 1, k == 6)
        row0 = pl.multiple_of(pr0 + k * POOL_CK, 8)

        @pl.when(jnp.logical_not(is_tail))
        def _():
            pltpu.sync_copy(h2_ref.at[c, pl.ds(row0, POOL_CK)], hbuf)

        @pl.when(is_tail)
        def _():
            pltpu.sync_copy(h2_ref.at[c, pl.ds(row0, POOL_TAIL)],
                            hbuf.at[pl.ds(0, POOL_TAIL)])

        rows_k = jnp.where(is_tail, POOL_TAIL, POOL_CK)

        def row_body(p, carry2):
            nl = (k * POOL_CK + p) * 4
            for q in range(4):
                bid = bbuf[pl.ds(nl + q, LANES)][0]
                for j in range(2):
                    hv = hbuf[p, pl.ds(q * D_HALF + j * LANES, LANES)]
                    slo = pl.ds(j * LANES, LANES)
                    outb[bid, slo] = jnp.maximum(outb[bid, slo], hv)
            return carry2

        lax.fori_loop(0, rows_k, row_body, 0)
        return carry

    lax.fori_loop(0, 7, ck_body, 0)
    pltpu.sync_copy(outb, pp_ref.at[c, s])


def _make_pool_kernel():
    mesh = plsc.VectorSubcoreMesh(core_axis_name="c", subcore_axis_name="s",
                                  num_cores=NC, num_subcores=NS)
    return pl.kernel(
        _pool_body,
        out_type=jax.ShapeDtypeStruct((NC, NS, N_GRAPHS, D_HALF), jnp.float32),
        mesh=mesh,
        compiler_params=pltpu.CompilerParams(use_tc_tiling_on_sc=False),
        scratch_types=[
            pltpu.VMEM((POOL_CK, 128), jnp.float32),
            pltpu.VMEM((POOL_ROWS * 4 + LANES,), jnp.int32),
            pltpu.VMEM((N_GRAPHS, D_HALF), jnp.float32),
            pltpu.SemaphoreType.DMA,
        ],
    )


def _emb_tables_body(x_ref, wemb_ref, bemb_ref, wd_ref, ws_ref,
                     h_ref, td_ref, ts_ref):
    hb = jnp.dot(x_ref[...], wemb_ref[...],
                 preferred_element_type=jnp.float32) + bemb_ref[...]
    h_ref[...] = hb
    for cc in range(NC):
        td_ref[cc] = jnp.dot(hb, wd_ref[cc], preferred_element_type=jnp.float32)
        ts_ref[cc] = jnp.dot(hb, ws_ref[cc], preferred_element_type=jnp.float32)


def _ea_body(ev_ref, od_ref, wea_ref, bea_ref, out_ref):
    ev = ev_ref[...]
    od = od_ref[...]
    for l in range(N_LAYERS):
        for cc in range(NC):
            w = wea_ref[l, cc]
            bb = bea_ref[l, cc]
            out_ref[l * NC + cc] = jnp.concatenate(
                [jnp.dot(ev, w, preferred_element_type=jnp.float32) + bb,
                 jnp.dot(od, w, preferred_element_type=jnp.float32) + bb],
                axis=1)


def _reduce_body(agg_ref, out_ref):
    i = pl.program_id(0)

    @pl.when(i == 0)
    def _():
        out_ref[...] = jnp.zeros_like(out_ref)

    a0 = agg_ref[0]
    a1 = agg_ref[1]
    s0 = jnp.sum(a0, axis=0)
    s1 = jnp.sum(a1, axis=0)
    q0 = jnp.sum(a0 * a0, axis=0)
    q1 = jnp.sum(a1 * a1, axis=0)
    row0 = jnp.concatenate([s0, s1])[None, :]
    row1 = jnp.concatenate([q0, q1])[None, :]
    pad = jnp.zeros((6, D_NODE), jnp.float32)
    out_ref[...] += jnp.concatenate([row0, row1, pad], axis=0)


def _bn_stats(sums_ref, gamma_ref, beta_ref):
    mu = sums_ref[0:1, :] * (1.0 / N_NODES)
    msq = sums_ref[1:2, :] * (1.0 / N_NODES)
    var = msq - mu * mu
    inv = gamma_ref[...] / jnp.sqrt(var + 1e-5)
    shift = beta_ref[...] - mu * inv
    return inv, shift


def _bn_tables_body(h_ref, agg_ref, sums_ref, gamma_ref, beta_ref,
                    wd_ref, ws_ref, hn_ref, td_ref, ts_ref):
    inv, shift = _bn_stats(sums_ref, gamma_ref, beta_ref)
    c0 = h_ref[:, 0:D_HALF] + agg_ref[0] * inv[:, 0:D_HALF] + shift[:, 0:D_HALF]
    c1 = h_ref[:, D_HALF:] + agg_ref[1] * inv[:, D_HALF:] + shift[:, D_HALF:]
    hn = jnp.concatenate([c0, c1], axis=1)
    hn_ref[...] = hn
    for cc in range(NC):
        td_ref[cc] = jnp.dot(hn, wd_ref[cc], preferred_element_type=jnp.float32)
        ts_ref[cc] = jnp.dot(hn, ws_ref[cc], preferred_element_type=jnp.float32)


def _bn_final_body(h_ref, agg_ref, sums_ref, gamma_ref, beta_ref, h2_ref):
    inv, shift = _bn_stats(sums_ref, gamma_ref, beta_ref)
    h2_ref[0] = (h_ref[:, 0:D_HALF] + agg_ref[0] * inv[:, 0:D_HALF]
                 + shift[:, 0:D_HALF])
    h2_ref[1] = (h_ref[:, D_HALF:] + agg_ref[1] * inv[:, D_HALF:]
                 + shift[:, D_HALF:])


def _head_body(pp_ref, wfc_ref, bfc_ref, wout_ref, bout_ref, out_ref):
    p0 = jnp.max(pp_ref[0], axis=0)
    p1 = jnp.max(pp_ref[1], axis=0)
    pooled = jnp.concatenate([p0, p1], axis=1)
    t = jnp.dot(pooled, wfc_ref[...],
                preferred_element_type=jnp.float32) + bfc_ref[...]
    sp = jnp.maximum(t, 0.0) + jnp.log(1.0 + jnp.exp(-jnp.abs(t)))
    out_ref[...] = jnp.dot(sp, wout_ref[...],
                           preferred_element_type=jnp.float32) + bout_ref[...]


def _row_spec(shape):
    nd = len(shape)
    return pl.BlockSpec(shape, lambda i: (0,) * nd)


def kernel(x, edge_index, edge_attr, batch, W_emb, b_emb, Wf, bf, Ws, bs,
           gamma, beta, W_fc, b_fc, W_out, b_out):
    f32 = jnp.float32
    src = edge_index[0]
    dst = edge_index[1]

    half = lambda w, c: w[:, c * D_HALF:(c + 1) * D_HALF]
    WD = jnp.stack([jnp.stack([
        jnp.concatenate([half(Wf[l][0:64], c), half(Ws[l][0:64], c)], axis=1)
        for c in range(NC)]) for l in range(N_LAYERS)])
    WS = jnp.stack([jnp.stack([
        jnp.concatenate([half(Wf[l][64:128], c), half(Ws[l][64:128], c)], axis=1)
        for c in range(NC)]) for l in range(N_LAYERS)])
    WEA = jnp.stack([jnp.stack([
        jnp.concatenate([half(Wf[l][128:144], c), half(Ws[l][128:144], c)],
                        axis=1)
        for c in range(NC)]) for l in range(N_LAYERS)])
    BEA = jnp.stack([jnp.stack([
        jnp.concatenate([half(bf[l][None], c)[0], half(bs[l][None], c)[0]])
        for c in range(NC)]) for l in range(N_LAYERS)])[:, :, None, :]

    zeros_n = jnp.zeros((N_PACK, 128), f32)

    E2 = N_EDGES // 2
    ea_pack = pl.pallas_call(
        _ea_body,
        grid=(_GRID_E // 2,),
        in_specs=[
            pl.BlockSpec((_BLK, 16), lambda i: (i, 0)),
            pl.BlockSpec((_BLK, 16), lambda i: (i, 0)),
            _row_spec((N_LAYERS, NC, 16, D_NODE)),
            _row_spec((N_LAYERS, NC, 1, D_NODE)),
        ],
        out_specs=pl.BlockSpec((N_LAYERS * NC, _BLK, 128),
                               lambda i: (0, i, 0)),
        out_shape=jax.ShapeDtypeStruct((N_LAYERS * NC, E2, 128), f32),
    )(edge_attr[0::2], edge_attr[1::2], WEA, BEA)

    h, td, ts = pl.pallas_call(
        _emb_tables_body,
        grid=(_GRID_N,),
        in_specs=[
            pl.BlockSpec((_BLK, 128), lambda i: (i, 0)),
            _row_spec((128, D_NODE)),
            _row_spec((1, D_NODE)),
            _row_spec((NC, D_NODE, D_NODE)),
            _row_spec((NC, D_NODE, D_NODE)),
        ],
        out_specs=[
            pl.BlockSpec((_BLK, D_NODE), lambda i: (i, 0)),
            pl.BlockSpec((NC, _BLK, D_NODE), lambda i: (0, i, 0)),
            pl.BlockSpec((NC, _BLK, D_NODE), lambda i: (0, i, 0)),
        ],
        out_shape=[
            jax.ShapeDtypeStruct((N_NODES, D_NODE), f32),
            jax.ShapeDtypeStruct((NC, N_NODES, D_NODE), f32),
            jax.ShapeDtypeStruct((NC, N_NODES, D_NODE), f32),
        ],
    )(x, W_emb, b_emb[None, :], WD[0], WS[0])

    reduce_call = pl.pallas_call(
        _reduce_body,
        grid=(_GRID_N,),
        in_specs=[pl.BlockSpec((NC, _BLK, D_HALF), lambda i: (0, i, 0))],
        out_specs=pl.BlockSpec((8, D_NODE), lambda i: (0, 0)),
        out_shape=jax.ShapeDtypeStruct((8, D_NODE), f32),
    )

    h2 = None
    for l in range(N_LAYERS):
        edge_call = _make_edge_kernel(l)
        agg2 = edge_call(td.reshape(NC * N_NODES, D_NODE),
                         ts.reshape(NC * N_NODES, D_NODE),
                         ea_pack, dst, src, zeros_n)
        agg2 = agg2.reshape(NC, N_NODES, D_HALF)
        sums = reduce_call(agg2)
        if l < N_LAYERS - 1:
            h, td, ts = pl.pallas_call(
                _bn_tables_body,
                grid=(_GRID_N,),
                in_specs=[
                    pl.BlockSpec((_BLK, D_NODE), lambda i: (i, 0)),
                    pl.BlockSpec((NC, _BLK, D_HALF), lambda i: (0, i, 0)),
                    _row_spec((8, D_NODE)),
                    _row_spec((1, D_NODE)),
                    _row_spec((1, D_NODE)),
                    _row_spec((NC, D_NODE, D_NODE)),
                    _row_spec((NC, D_NODE, D_NODE)),
                ],
                out_specs=[
                    pl.BlockSpec((_BLK, D_NODE), lambda i: (i, 0)),
                    pl.BlockSpec((NC, _BLK, D_NODE), lambda i: (0, i, 0)),
                    pl.BlockSpec((NC, _BLK, D_NODE), lambda i: (0, i, 0)),
                ],
                out_shape=[
                    jax.ShapeDtypeStruct((N_NODES, D_NODE), f32),
                    jax.ShapeDtypeStruct((NC, N_NODES, D_NODE), f32),
                    jax.ShapeDtypeStruct((NC, N_NODES, D_NODE), f32),
                ],
            )(h, agg2, sums, gamma[l][None, :], beta[l][None, :],
              WD[l + 1], WS[l + 1])
        else:
            h2 = pl.pallas_call(
                _bn_final_body,
                grid=(_GRID_N,),
                in_specs=[
                    pl.BlockSpec((_BLK, D_NODE), lambda i: (i, 0)),
                    pl.BlockSpec((NC, _BLK, D_HALF), lambda i: (0, i, 0)),
                    _row_spec((8, D_NODE)),
                    _row_spec((1, D_NODE)),
                    _row_spec((1, D_NODE)),
                ],
                out_specs=pl.BlockSpec((NC, _BLK, D_HALF), lambda i: (0, i, 0)),
                out_shape=jax.ShapeDtypeStruct((NC, N_NODES, D_HALF), f32),
            )(h, agg2, sums, gamma[l][None, :], beta[l][None, :])

    pool_call = _make_pool_kernel()
    pp = pool_call(h2.reshape(NC, N_PACK, 128), batch)

    out = pl.pallas_call(
        _head_body,
        in_specs=[
            pl.BlockSpec((NC, NS, N_GRAPHS, D_HALF), lambda: (0, 0, 0, 0)),
            pl.BlockSpec((D_NODE, 128), lambda: (0, 0)),
            pl.BlockSpec((1, 128), lambda: (0, 0)),
            pl.BlockSpec((128, 1), lambda: (0, 0)),
            pl.BlockSpec((1, 1), lambda: (0, 0)),
        ],
        out_specs=pl.BlockSpec((N_GRAPHS, 1), lambda: (0, 0)),
        out_shape=jax.ShapeDtypeStruct((N_GRAPHS, 1), f32),
    )(pp, W_fc, b_fc[None, :], W_out, b_out[None, :])
    return out


# speedup vs baseline: 3.1386x; 1.2372x over previous
import jax
import jax.numpy as jnp
from jax import lax
from jax.experimental import pallas as pl
from jax.experimental.pallas import tpu as pltpu
from jax.experimental.pallas import tpu_sc as plsc

N_NODES = 50000
N_EDGES = 800000
D_NODE = 64
D_HALF = 32
N_LAYERS = 3
N_GRAPHS = 128

NC = 2
NS = 16
LANES = 16

CHUNK = 32
N_CHUNKS = N_EDGES // CHUNK
ROWS_PER_TILE = 3120
ROWS_LAST = N_NODES - (NS - 1) * ROWS_PER_TILE

N_PACK = N_NODES // 4
P_ROWS_PER_TILE = 768
P_ROWS_LAST = N_PACK - (NS - 1) * P_ROWS_PER_TILE

POOL_ROWS = 784
POOL_LAST = N_PACK - (NS - 1) * POOL_ROWS
POOL_CK = 112
POOL_TAIL = POOL_LAST - 6 * POOL_CK

_BLK = 1000
_GRID_N = N_NODES // _BLK
_GRID_E = N_EDGES // _BLK


def _recip(x):
    r = plsc.bitcast(jnp.asarray(0x7EF311C3, jnp.int32)
                     - plsc.bitcast(x, jnp.int32), jnp.float32)
    r = r * (2.0 - x * r)
    r = r * (2.0 - x * r)
    r = r * (2.0 - x * r)
    return r


def _sigmoid16(u):
    eu = jnp.exp(-jnp.abs(u))
    s = _recip(1.0 + eu)
    return jnp.where(u >= 0.0, s, eu * s)


def _softplus16(v):
    ev = jnp.exp(-jnp.abs(v))
    t = ev * _recip(2.0 + ev)
    t2 = t * t
    p = 2.0 * t * (1.0 + t2 * (1.0 / 3.0 + t2 * (0.2 + t2 * (1.0 / 7.0))))
    return jnp.maximum(v, 0.0) + p


EA_ROWS = CHUNK // 2
PAIRS_TOTAL = N_CHUNKS // 2


def _make_edge_kernel(layer):
    ea_plane0 = layer * NC

    def body(td_ref, ts_ref, ea_ref, dst_ref, src_ref, z_ref, agg_ref,
             agg_sh, gdst, gsrc, scidx, dbuf, sbuf, eabuf, mbuf,
             sg0, sg1, si0, si1, se0, se1, sc0, sc1):
        c = lax.axis_index("c")
        s = lax.axis_index("s")
        c_n = c * N_NODES
        c_p = c * N_PACK
        r0 = pl.multiple_of(s * P_ROWS_PER_TILE, 8)
        sg = (sg0, sg1)
        si = (si0, si1)
        se = (se0, se1)
        sc = (sc0, sc1)

        @pl.when(s < NS - 1)
        def _():
            pltpu.sync_copy(z_ref.at[pl.ds(r0, P_ROWS_PER_TILE)],
                            agg_sh.at[pl.ds(r0, P_ROWS_PER_TILE)])

        @pl.when(s == NS - 1)
        def _():
            pltpu.sync_copy(z_ref.at[pl.ds(r0, P_ROWS_LAST)],
                            agg_sh.at[pl.ds(r0, P_ROWS_LAST)])

        plsc.subcore_barrier()

        base_pairs = PAIRS_TOTAL // NS
        extra = PAIRS_TOTAL - NS * base_pairs
        p0 = s * base_pairs + jnp.minimum(s, extra)
        pcnt = base_pairs + jnp.where(s < extra, 1, 0)
        k0 = p0 * 2
        zv = jnp.zeros((LANES,), jnp.float32)

        def slot(buf, b, n):
            return buf.at[pl.ds(b * n, n)]

        def issue_idx_ea(i, b):
            k = k0 + i
            e0 = pl.multiple_of(k * CHUNK, CHUNK)
            pltpu.async_copy(dst_ref.at[pl.ds(e0, CHUNK)],
                             slot(gdst, b, CHUNK), si[b])
            pltpu.async_copy(src_ref.at[pl.ds(e0, CHUNK)],
                             slot(gsrc, b, CHUNK), si[b])
            pltpu.async_copy(
                ea_ref.at[ea_plane0 + c, pl.ds(k * EA_ROWS, EA_ROWS)],
                slot(eabuf, b, EA_ROWS), se[b])

        def wait_idx(b):
            pltpu.make_async_copy(dst_ref.at[pl.ds(0, CHUNK)],
                                  slot(gdst, b, CHUNK), si[b]).wait()
            pltpu.make_async_copy(src_ref.at[pl.ds(0, CHUNK)],
                                  slot(gsrc, b, CHUNK), si[b]).wait()

        def wait_ea(b):
            pltpu.make_async_copy(ea_ref.at[0, pl.ds(0, EA_ROWS)],
                                  slot(eabuf, b, EA_ROWS), se[b]).wait()

        def modify_idx(b):
            for q in range(CHUNK // LANES):
                sl = pl.ds(b * CHUNK + q * LANES, LANES)
                gdst[sl] = gdst[sl] + c_n
                gsrc[sl] = gsrc[sl] + c_n

        def issue_gather(b):
            pltpu.async_copy(td_ref.at[slot(gdst, b, CHUNK)],
                             slot(dbuf, b, CHUNK), sg[b])
            pltpu.async_copy(ts_ref.at[slot(gsrc, b, CHUNK)],
                             slot(sbuf, b, CHUNK), sg[b])

        def wait_gather(b):
            pltpu.make_async_copy(td_ref.at[slot(gdst, b, CHUNK)],
                                  slot(dbuf, b, CHUNK), sg[b]).wait()
            pltpu.make_async_copy(ts_ref.at[slot(gsrc, b, CHUNK)],
                                  slot(sbuf, b, CHUNK), sg[b]).wait()

        def compute(b):
            for q in range(CHUNK // LANES):
                sl = pl.ds(b * CHUNK + q * LANES, LANES)
                scidx[b, pl.ds(q * LANES, LANES)] = (gdst[sl] >> 2) - c_p

            lanes = lax.broadcasted_iota(jnp.int32, (LANES,), 0)

            @plsc.parallel_loop(0, CHUNK, unroll=4)
            def _(e):
                row = b * CHUNK + e
                rowv = jnp.zeros((LANES,), jnp.int32) + row
                dv = plsc.load_gather(gdst, [rowv])
                offv = (dv & 3) * D_HALF + lanes
                earow = b * EA_ROWS + (e >> 1)
                ecb = (e & 1) * D_NODE
                for q in range(8):
                    mbuf[row, pl.ds(q * LANES, LANES)] = zv
                for j in range(2):
                    slu = pl.ds(j * LANES, LANES)
                    slv = pl.ds(D_HALF + j * LANES, LANES)
                    u = (dbuf[row, slu] + sbuf[row, slu]
                         + eabuf[earow, pl.ds(ecb + j * LANES, LANES)])
                    v = (dbuf[row, slv] + sbuf[row, slv]
                         + eabuf[earow,
                                 pl.ds(ecb + D_HALF + j * LANES, LANES)])
                    m = _sigmoid16(u) * _softplus16(v)
                    plsc.store_scatter(mbuf, [rowv, offv + (j * LANES)], m)

        def issue_scatter(b):
            pltpu.async_copy(slot(mbuf, b, CHUNK), agg_sh.at[scidx.at[b]],
                             sc[b], add=True)

        def wait_scatter(b):
            pltpu.make_async_copy(slot(mbuf, b, CHUNK),
                                  agg_sh.at[scidx.at[b]], sc[b]).wait()

        issue_idx_ea(0, 0)
        issue_idx_ea(1, 1)
        wait_idx(0)
        modify_idx(0)
        issue_gather(0)

        def pair_body(kk, carry):
            for b in (0, 1):
                i = kk * 2 + b
                ob = 1 - b
                if b == 0:
                    wait_idx(ob)
                    modify_idx(ob)
                    issue_gather(ob)
                else:
                    @pl.when(kk < pcnt - 1)
                    def _():
                        wait_idx(ob)
                        modify_idx(ob)
                        issue_gather(ob)
                wait_gather(b)

                @pl.when(kk >= 1)
                def _():
                    wait_scatter(b)

                wait_ea(b)
                compute(b)
                issue_scatter(b)

                @pl.when(kk < pcnt - 1)
                def _():
                    issue_idx_ea(i + 2, b)
            return carry

        lax.fori_loop(0, pcnt, pair_body, 0)
        wait_scatter(0)
        wait_scatter(1)
        plsc.subcore_barrier()

        @pl.when(s < NS - 1)
        def _():
            pltpu.sync_copy(agg_sh.at[pl.ds(r0, P_ROWS_PER_TILE)],
                            agg_ref.at[c, pl.ds(r0, P_ROWS_PER_TILE)])

        @pl.when(s == NS - 1)
        def _():
            pltpu.sync_copy(agg_sh.at[pl.ds(r0, P_ROWS_LAST)],
                            agg_ref.at[c, pl.ds(r0, P_ROWS_LAST)])

    mesh = plsc.VectorSubcoreMesh(core_axis_name="c", subcore_axis_name="s",
                                  num_cores=NC, num_subcores=NS)
    return pl.kernel(
        body,
        out_type=jax.ShapeDtypeStruct((NC, N_PACK, 128), jnp.float32),
        mesh=mesh,
        compiler_params=pltpu.CompilerParams(use_tc_tiling_on_sc=False,
                                             needs_layout_passes=False),
        scratch_types=[
            pltpu.VMEM_SHARED((N_PACK, 128), jnp.float32),
            pltpu.VMEM((2 * CHUNK + LANES,), jnp.int32),
            pltpu.VMEM((2 * CHUNK,), jnp.int32),
            pltpu.VMEM((2, CHUNK), jnp.int32),
            pltpu.VMEM((2 * CHUNK, D_NODE), jnp.float32),
            pltpu.VMEM((2 * CHUNK, D_NODE), jnp.float32),
            pltpu.VMEM((2 * EA_ROWS, 128), jnp.float32),
            pltpu.VMEM((2 * CHUNK, 128), jnp.float32),
            pltpu.SemaphoreType.DMA,
            pltpu.SemaphoreType.DMA,
            pltpu.SemaphoreType.DMA,
            pltpu.SemaphoreType.DMA,
            pltpu.SemaphoreType.DMA,
            pltpu.SemaphoreType.DMA,
            pltpu.SemaphoreType.DMA,
            pltpu.SemaphoreType.DMA,
        ],
    )


def _pool_body(h2_ref, batch_ref, pp_ref, hbuf, bbuf, outb, sem):
    c = lax.axis_index("c")
    s = lax.axis_index("s")
    pr0 = pl.multiple_of(s * POOL_ROWS, 8)
    nb0 = pl.multiple_of(s * POOL_ROWS * 4, 8)
    neg = jnp.full((LANES,), -jnp.inf, jnp.float32)

    @pl.when(s < NS - 1)
    def _():
        pltpu.sync_copy(batch_ref.at[pl.ds(nb0, POOL_ROWS * 4)],
                        bbuf.at[pl.ds(0, POOL_ROWS * 4)])

    @pl.when(s == NS - 1)
    def _():
        pltpu.sync_copy(batch_ref.at[pl.ds(nb0, POOL_LAST * 4)],
                        bbuf.at[pl.ds(0, POOL_LAST * 4)])

    def init_body(i, carry):
        for j in range(2):
            outb[i, pl.ds(j * LANES, LANES)] = neg
        return carry

    lax.fori_loop(0, N_GRAPHS, init_body, 0)

    def ck_body(k, carry):
        is_tail = jnp.logical_and(s == NS - 1, k == 6)
        row0 = pl.multiple_of(pr0 + k * POOL_CK, 8)

        @pl.when(jnp.logical_not(is_tail))
        def _():
            pltpu.sync_copy(h2_ref.at[c, pl.ds(row0, POOL_CK)], hbuf)

        @pl.when(is_tail)
        def _():
            pltpu.sync_copy(h2_ref.at[c, pl.ds(row0, POOL_TAIL)],
                            hbuf.at[pl.ds(0, POOL_TAIL)])

        rows_k = jnp.where(is_tail, POOL_TAIL, POOL_CK)

        def row_body(p, carry2):
            nl = (k * POOL_CK + p) * 4
            for q in range(4):
                bid = bbuf[pl.ds(nl + q, LANES)][0]
                for j in range(2):
                    hv = hbuf[p, pl.ds(q * D_HALF + j * LANES, LANES)]
                    slo = pl.ds(j * LANES, LANES)
                    outb[bid, slo] = jnp.maximum(outb[bid, slo], hv)
            return carry2

        lax.fori_loop(0, rows_k, row_body, 0)
        return carry

    lax.fori_loop(0, 7, ck_body, 0)
    pltpu.sync_copy(outb, pp_ref.at[c, s])


def _make_pool_kernel():
    mesh = plsc.VectorSubcoreMesh(core_axis_name="c", subcore_axis_name="s",
                                  num_cores=NC, num_subcores=NS)
    return pl.kernel(
        _pool_body,
        out_type=jax.ShapeDtypeStruct((NC, NS, N_GRAPHS, D_HALF), jnp.float32),
        mesh=mesh,
        compiler_params=pltpu.CompilerParams(use_tc_tiling_on_sc=False),
        scratch_types=[
            pltpu.VMEM((POOL_CK, 128), jnp.float32),
            pltpu.VMEM((POOL_ROWS * 4 + LANES,), jnp.int32),
            pltpu.VMEM((N_GRAPHS, D_HALF), jnp.float32),
            pltpu.SemaphoreType.DMA,
        ],
    )


def _emb_tables_body(x_ref, wemb_ref, bemb_ref, wd_ref, ws_ref,
                     h_ref, td_ref, ts_ref):
    hb = jnp.dot(x_ref[...], wemb_ref[...],
                 preferred_element_type=jnp.float32) + bemb_ref[...]
    h_ref[...] = hb
    for cc in range(NC):
        td_ref[cc] = jnp.dot(hb, wd_ref[cc], preferred_element_type=jnp.float32)
        ts_ref[cc] = jnp.dot(hb, ws_ref[cc], preferred_element_type=jnp.float32)


def _ea_body(ea2_ref, wea_ref, bea_ref, out_ref):
    blk = ea2_ref[...]
    n = blk.shape[0]
    ev = blk[:, 0:16]
    od = blk[:, 16:32]
    z = jnp.dot(jnp.concatenate([ev, od], axis=0), wea_ref[...],
                preferred_element_type=jnp.float32) + bea_ref[...]
    for lc in range(N_LAYERS * NC):
        sl = slice(lc * D_NODE, (lc + 1) * D_NODE)
        out_ref[lc] = jnp.concatenate([z[0:n, sl], z[n:2 * n, sl]], axis=1)


def _reduce_body(agg_ref, out_ref):
    i = pl.program_id(0)

    @pl.when(i == 0)
    def _():
        out_ref[...] = jnp.zeros_like(out_ref)

    a0 = agg_ref[0]
    a1 = agg_ref[1]
    s0 = jnp.sum(a0, axis=0)
    s1 = jnp.sum(a1, axis=0)
    q0 = jnp.sum(a0 * a0, axis=0)
    q1 = jnp.sum(a1 * a1, axis=0)
    row0 = jnp.concatenate([s0, s1])[None, :]
    row1 = jnp.concatenate([q0, q1])[None, :]
    pad = jnp.zeros((6, D_NODE), jnp.float32)
    out_ref[...] += jnp.concatenate([row0, row1, pad], axis=0)


def _bn_stats(sums_ref, gamma_ref, beta_ref):
    mu = sums_ref[0:1, :] * (1.0 / N_NODES)
    msq = sums_ref[1:2, :] * (1.0 / N_NODES)
    var = msq - mu * mu
    inv = gamma_ref[...] / jnp.sqrt(var + 1e-5)
    shift = beta_ref[...] - mu * inv
    return inv, shift


def _bn_tables_body(h_ref, agg_ref, sums_ref, gamma_ref, beta_ref,
                    wd_ref, ws_ref, hn_ref, td_ref, ts_ref):
    inv, shift = _bn_stats(sums_ref, gamma_ref, beta_ref)
    c0 = h_ref[:, 0:D_HALF] + agg_ref[0] * inv[:, 0:D_HALF] + shift[:, 0:D_HALF]
    c1 = h_ref[:, D_HALF:] + agg_ref[1] * inv[:, D_HALF:] + shift[:, D_HALF:]
    hn = jnp.concatenate([c0, c1], axis=1)
    hn_ref[...] = hn
    for cc in range(NC):
        td_ref[cc] = jnp.dot(hn, wd_ref[cc], preferred_element_type=jnp.float32)
        ts_ref[cc] = jnp.dot(hn, ws_ref[cc], preferred_element_type=jnp.float32)


def _bn_final_body(h_ref, agg_ref, sums_ref, gamma_ref, beta_ref, h2_ref):
    inv, shift = _bn_stats(sums_ref, gamma_ref, beta_ref)
    h2_ref[0] = (h_ref[:, 0:D_HALF] + agg_ref[0] * inv[:, 0:D_HALF]
                 + shift[:, 0:D_HALF])
    h2_ref[1] = (h_ref[:, D_HALF:] + agg_ref[1] * inv[:, D_HALF:]
                 + shift[:, D_HALF:])


def _head_body(pp_ref, wfc_ref, bfc_ref, wout_ref, bout_ref, out_ref):
    p0 = jnp.max(pp_ref[0], axis=0)
    p1 = jnp.max(pp_ref[1], axis=0)
    pooled = jnp.concatenate([p0, p1], axis=1)
    t = jnp.dot(pooled, wfc_ref[...],
                preferred_element_type=jnp.float32) + bfc_ref[...]
    sp = jnp.maximum(t, 0.0) + jnp.log(1.0 + jnp.exp(-jnp.abs(t)))
    out_ref[...] = jnp.dot(sp, wout_ref[...],
                           preferred_element_type=jnp.float32) + bout_ref[...]


def _row_spec(shape):
    nd = len(shape)
    return pl.BlockSpec(shape, lambda i: (0,) * nd)


def kernel(x, edge_index, edge_attr, batch, W_emb, b_emb, Wf, bf, Ws, bs,
           gamma, beta, W_fc, b_fc, W_out, b_out):
    f32 = jnp.float32
    src = edge_index[0]
    dst = edge_index[1]

    half = lambda w, c: w[:, c * D_HALF:(c + 1) * D_HALF]
    WD = jnp.stack([jnp.stack([
        jnp.concatenate([half(Wf[l][0:64], c), half(Ws[l][0:64], c)], axis=1)
        for c in range(NC)]) for l in range(N_LAYERS)])
    WS = jnp.stack([jnp.stack([
        jnp.concatenate([half(Wf[l][64:128], c), half(Ws[l][64:128], c)], axis=1)
        for c in range(NC)]) for l in range(N_LAYERS)])
    WEA = jnp.stack([jnp.stack([
        jnp.concatenate([half(Wf[l][128:144], c), half(Ws[l][128:144], c)],
                        axis=1)
        for c in range(NC)]) for l in range(N_LAYERS)])
    BEA = jnp.stack([jnp.stack([
        jnp.concatenate([half(bf[l][None], c)[0], half(bs[l][None], c)[0]])
        for c in range(NC)]) for l in range(N_LAYERS)])[:, :, None, :]

    zeros_n = jnp.zeros((N_PACK, 128), f32)

    E2 = N_EDGES // 2
    WEA384 = jnp.concatenate(
        [WEA[l, cc] for l in range(N_LAYERS) for cc in range(NC)], axis=1)
    BEA384 = jnp.concatenate(
        [BEA[l, cc] for l in range(N_LAYERS) for cc in range(NC)], axis=1)
    ea_pack = pl.pallas_call(
        _ea_body,
        grid=(_GRID_E // 2,),
        in_specs=[
            pl.BlockSpec((_BLK, 32), lambda i: (i, 0)),
            _row_spec((16, N_LAYERS * NC * D_NODE)),
            _row_spec((1, N_LAYERS * NC * D_NODE)),
        ],
        out_specs=pl.BlockSpec((N_LAYERS * NC, _BLK, 128),
                               lambda i: (0, i, 0)),
        out_shape=jax.ShapeDtypeStruct((N_LAYERS * NC, E2, 128), f32),
    )(edge_attr.reshape(E2, 32), WEA384, BEA384)

    h, td, ts = pl.pallas_call(
        _emb_tables_body,
        grid=(_GRID_N,),
        in_specs=[
            pl.BlockSpec((_BLK, 128), lambda i: (i, 0)),
            _row_spec((128, D_NODE)),
            _row_spec((1, D_NODE)),
            _row_spec((NC, D_NODE, D_NODE)),
            _row_spec((NC, D_NODE, D_NODE)),
        ],
        out_specs=[
            pl.BlockSpec((_BLK, D_NODE), lambda i: (i, 0)),
            pl.BlockSpec((NC, _BLK, D_NODE), lambda i: (0, i, 0)),
            pl.BlockSpec((NC, _BLK, D_NODE), lambda i: (0, i, 0)),
        ],
        out_shape=[
            jax.ShapeDtypeStruct((N_NODES, D_NODE), f32),
            jax.ShapeDtypeStruct((NC, N_NODES, D_NODE), f32),
            jax.ShapeDtypeStruct((NC, N_NODES, D_NODE), f32),
        ],
    )(x, W_emb, b_emb[None, :], WD[0], WS[0])

    reduce_call = pl.pallas_call(
        _reduce_body,
        grid=(_GRID_N,),
        in_specs=[pl.BlockSpec((NC, _BLK, D_HALF), lambda i: (0, i, 0))],
        out_specs=pl.BlockSpec((8, D_NODE), lambda i: (0, 0)),
        out_shape=jax.ShapeDtypeStruct((8, D_NODE), f32),
    )

    h2 = None
    for l in range(N_LAYERS):
        edge_call = _make_edge_kernel(l)
        agg2 = edge_call(td.reshape(NC * N_NODES, D_NODE),
                         ts.reshape(NC * N_NODES, D_NODE),
                         ea_pack, dst, src, zeros_n)
        agg2 = agg2.reshape(NC, N_NODES, D_HALF)
        sums = reduce_call(agg2)
        if l < N_LAYERS - 1:
            h, td, ts = pl.pallas_call(
                _bn_tables_body,
                grid=(_GRID_N,),
                in_specs=[
                    pl.BlockSpec((_BLK, D_NODE), lambda i: (i, 0)),
                    pl.BlockSpec((NC, _BLK, D_HALF), lambda i: (0, i, 0)),
                    _row_spec((8, D_NODE)),
                    _row_spec((1, D_NODE)),
                    _row_spec((1, D_NODE)),
                    _row_spec((NC, D_NODE, D_NODE)),
                    _row_spec((NC, D_NODE, D_NODE)),
                ],
                out_specs=[
                    pl.BlockSpec((_BLK, D_NODE), lambda i: (i, 0)),
                    pl.BlockSpec((NC, _BLK, D_NODE), lambda i: (0, i, 0)),
                    pl.BlockSpec((NC, _BLK, D_NODE), lambda i: (0, i, 0)),
                ],
                out_shape=[
                    jax.ShapeDtypeStruct((N_NODES, D_NODE), f32),
                    jax.ShapeDtypeStruct((NC, N_NODES, D_NODE), f32),
                    jax.ShapeDtypeStruct((NC, N_NODES, D_NODE), f32),
                ],
            )(h, agg2, sums, gamma[l][None, :], beta[l][None, :],
              WD[l + 1], WS[l + 1])
        else:
            h2 = pl.pallas_call(
                _bn_final_body,
                grid=(_GRID_N,),
                in_specs=[
                    pl.BlockSpec((_BLK, D_NODE), lambda i: (i, 0)),
                    pl.BlockSpec((NC, _BLK, D_HALF), lambda i: (0, i, 0)),
                    _row_spec((8, D_NODE)),
                    _row_spec((1, D_NODE)),
                    _row_spec((1, D_NODE)),
                ],
                out_specs=pl.BlockSpec((NC, _BLK, D_HALF), lambda i: (0, i, 0)),
                out_shape=jax.ShapeDtypeStruct((NC, N_NODES, D_HALF), f32),
            )(h, agg2, sums, gamma[l][None, :], beta[l][None, :])

    pool_call = _make_pool_kernel()
    pp = pool_call(h2.reshape(NC, N_PACK, 128), batch)

    out = pl.pallas_call(
        _head_body,
        in_specs=[
            pl.BlockSpec((NC, NS, N_GRAPHS, D_HALF), lambda: (0, 0, 0, 0)),
            pl.BlockSpec((D_NODE, 128), lambda: (0, 0)),
            pl.BlockSpec((1, 128), lambda: (0, 0)),
            pl.BlockSpec((128, 1), lambda: (0, 0)),
            pl.BlockSpec((1, 1), lambda: (0, 0)),
        ],
        out_specs=pl.BlockSpec((N_GRAPHS, 1), lambda: (0, 0)),
        out_shape=jax.ShapeDtypeStruct((N_GRAPHS, 1), f32),
    )(pp, W_fc, b_fc[None, :], W_out, b_out[None, :])
    return out
